# Initial kernel scaffold; baseline (speedup 1.0000x reference)
import jax
import jax.numpy as jnp
from jax.experimental import pallas as pl


def kernel(x, edge_index, W_lin, b_lin, Wc1, bc1, Wc2, bc2, W_ih, W_hh, b_ih, b_hh, Wp1, bp1, g1, bn1, Wp2, bp2, g2, bn2, Wp3, bp3):
    raise NotImplementedError("write your pallas kernel here")


# speedup vs baseline: 24.5520x; 24.5520x over previous
import functools

import jax
import jax.numpy as jnp
from jax import lax
from jax.experimental import pallas as pl
from jax.experimental.pallas import tpu as pltpu
from jax.experimental.pallas import tpu_sc as plsc

N = 10000
E = 320000
F = 128
H = 128

NP = 10240
NC = 2
NS = 16
NW = NC * NS
EPW = E // NW
CHUNK = 80
NCHUNKS = EPW // CHUNK
NBUF = 4
NGROUP = NCHUNKS // NBUF
RPT = NP // NS
BLK = 512
GRID = NP // BLK

@functools.cache
def _mesh():
    return plsc.VectorSubcoreMesh(
        core_axis_name="c", subcore_axis_name="s", num_cores=NC, num_subcores=NS
    )


def _deg_body(dst_hbm, out_hbm, deg_v, idx_v):
    c = lax.axis_index("c")
    s = lax.axis_index("s")
    wid = s * NC + c

    zeros16 = jnp.zeros((16,), jnp.float32)

    def zloop(i, carry):
        deg_v[pl.ds(i * 16, 16)] = zeros16
        return carry

    lax.fori_loop(0, NP // 16, zloop, 0)

    pltpu.sync_copy(dst_hbm.at[wid], idx_v)
    ones16 = jnp.ones((16,), jnp.float32)

    def body(i, carry):
        idx = idx_v[pl.ds(i * 16, 16)]
        plsc.addupdate_scatter(deg_v, [idx], ones16)
        return carry

    lax.fori_loop(0, EPW // 16, body, 0)
    pltpu.sync_copy(deg_v, out_hbm.at[wid])


@functools.cache
def _deg_kernel():
    return pl.kernel(
        _deg_body,
        out_type=jax.ShapeDtypeStruct((NW, NP), jnp.float32),
        mesh=_mesh(),
        scratch_types=[
            pltpu.VMEM((NP,), jnp.float32),
            pltpu.VMEM((EPW,), jnp.int32),
        ],
        compiler_params=pltpu.CompilerParams(needs_layout_passes=False),
    )


def _conv_body(g_hbm, src_hbm, dst_hbm, zc_hbm, out_hbm,
               acc_sh, src_v, dst_v, rows_v, zero_v, gsem, ssem):
    c = lax.axis_index("c")
    s = lax.axis_index("s")
    wid = s * NC + c

    pltpu.sync_copy(zc_hbm, zero_v)

    def zacc(i, carry):
        pltpu.sync_copy(zero_v, acc_sh.at[pl.ds((i * NS + s) * 16, 16)])
        return carry

    lax.fori_loop(0, RPT // 16, zacc, 0)
    plsc.subcore_barrier()

    def group(gi, carry):
        jb = gi * NBUF
        pltpu.sync_copy(src_hbm.at[wid, pl.ds(jb, NBUF)], src_v)
        pltpu.sync_copy(dst_hbm.at[wid, pl.ds(jb, NBUF)], dst_v)
        for b in range(NBUF):
            pltpu.async_copy(g_hbm.at[src_v.at[b]], rows_v.at[b], gsem)
        for b in range(NBUF):
            pltpu.make_async_copy(
                g_hbm.at[src_v.at[b]], rows_v.at[b], gsem
            ).wait()
            pltpu.async_copy(
                rows_v.at[b], acc_sh.at[dst_v.at[b]], ssem, add=True
            )
        for b in range(NBUF):
            pltpu.make_async_copy(
                rows_v.at[b], acc_sh.at[dst_v.at[b]], ssem
            ).wait()
        return carry

    lax.fori_loop(0, NGROUP, group, 0)

    for j in range(NGROUP * NBUF, NCHUNKS):
        pltpu.sync_copy(src_hbm.at[wid, pl.ds(j, 1)], src_v.at[pl.ds(0, 1)])
        pltpu.sync_copy(dst_hbm.at[wid, pl.ds(j, 1)], dst_v.at[pl.ds(0, 1)])
        pltpu.async_copy(g_hbm.at[src_v.at[0]], rows_v.at[0], gsem).wait()
        pltpu.async_copy(
            rows_v.at[0], acc_sh.at[dst_v.at[0]], ssem, add=True
        ).wait()
    plsc.subcore_barrier()

    row0 = s * RPT
    pltpu.sync_copy(acc_sh.at[pl.ds(row0, RPT)], out_hbm.at[c, pl.ds(row0, RPT)])


@functools.cache
def _conv_kernel():
    return pl.kernel(
        _conv_body,
        out_type=jax.ShapeDtypeStruct((NC, NP, H), jnp.float32),
        mesh=_mesh(),
        scratch_types=[
            pltpu.VMEM_SHARED((NP, H), jnp.float32),
            pltpu.VMEM((NBUF, CHUNK), jnp.int32),
            pltpu.VMEM((NBUF, CHUNK), jnp.int32),
            pltpu.VMEM((NBUF, CHUNK, H), jnp.float32),
            pltpu.VMEM((16, H), jnp.float32),
            pltpu.SemaphoreType.DMA,
            pltpu.SemaphoreType.DMA,
        ],
        compiler_params=pltpu.CompilerParams(use_tc_tiling_on_sc=False),
    )


def _dinv_from_partials(degp):
    deg = jnp.sum(degp, axis=0) + 1.0
    return lax.rsqrt(deg)[:, None]


def _layer_norm_tc(t, g, b):
    mu = jnp.mean(t, axis=-1, keepdims=True)
    v = jnp.mean((t - mu) ** 2, axis=-1, keepdims=True)
    return (t - mu) / jnp.sqrt(v + 1e-5) * g + b


def _stage_a_body(degp_ref, x_ref, wlin_ref, blin_ref, wc1_ref, hw1_ref, g1_ref):
    dinv = _dinv_from_partials(degp_ref[...])
    h0 = jnp.maximum(x_ref[...] @ wlin_ref[...] + blin_ref[...], 0.0)
    hw1 = h0 @ wc1_ref[...]
    hw1_ref[...] = hw1
    g1_ref[...] = hw1 * dinv


_stage_a = pl.pallas_call(
    _stage_a_body,
    grid=(GRID,),
    in_specs=[
        pl.BlockSpec((NW, BLK), lambda i: (0, i)),
        pl.BlockSpec((BLK, F), lambda i: (i, 0)),
        pl.BlockSpec((F, H), lambda i: (0, 0)),
        pl.BlockSpec((1, H), lambda i: (0, 0)),
        pl.BlockSpec((H, H), lambda i: (0, 0)),
    ],
    out_specs=[
        pl.BlockSpec((BLK, H), lambda i: (i, 0)),
        pl.BlockSpec((BLK, H), lambda i: (i, 0)),
    ],
    out_shape=[
        jax.ShapeDtypeStruct((NP, H), jnp.float32),
        jax.ShapeDtypeStruct((NP, H), jnp.float32),
    ],
)


def _stage_b_body(degp_ref, acc_ref, hw1_ref, bc1_ref, wc2_ref, hw2_ref, g2_ref):
    dinv = _dinv_from_partials(degp_ref[...])
    a = acc_ref[...]
    conv = dinv * (a[0] + a[1]) + (dinv * dinv) * hw1_ref[...] + bc1_ref[...]
    h1 = jnp.maximum(conv, 0.0)
    hw2 = h1 @ wc2_ref[...]
    hw2_ref[...] = hw2
    g2_ref[...] = hw2 * dinv


_stage_b = pl.pallas_call(
    _stage_b_body,
    grid=(GRID,),
    in_specs=[
        pl.BlockSpec((NW, BLK), lambda i: (0, i)),
        pl.BlockSpec((NC, BLK, H), lambda i: (0, i, 0)),
        pl.BlockSpec((BLK, H), lambda i: (i, 0)),
        pl.BlockSpec((1, H), lambda i: (0, 0)),
        pl.BlockSpec((H, H), lambda i: (0, 0)),
    ],
    out_specs=[
        pl.BlockSpec((BLK, H), lambda i: (i, 0)),
        pl.BlockSpec((BLK, H), lambda i: (i, 0)),
    ],
    out_shape=[
        jax.ShapeDtypeStruct((NP, H), jnp.float32),
        jax.ShapeDtypeStruct((NP, H), jnp.float32),
    ],
)


def _stage_c_body(degp_ref, acc_ref, hw2_ref, bc2_ref, wih_ref, bih_ref, bhh_ref,
                  wp1_ref, bp1_ref, g1_ref, bn1_ref,
                  wp2_ref, bp2_ref, g2_ref, bn2_ref,
                  wp3_ref, bp3_ref, y_ref):
    dinv = _dinv_from_partials(degp_ref[...])
    a = acc_ref[...]
    conv = dinv * (a[0] + a[1]) + (dinv * dinv) * hw2_ref[...] + bc2_ref[...]
    h2 = jnp.maximum(conv, 0.0)
    gi = h2 @ wih_ref[...] + bih_ref[...]
    bhh = bhh_ref[...]
    r = jax.nn.sigmoid(gi[:, :H] + bhh[:, :H])
    z = jax.nn.sigmoid(gi[:, H:2 * H] + bhh[:, H:2 * H])
    n = jnp.tanh(gi[:, 2 * H:] + r * bhh[:, 2 * H:])
    hcur = (1.0 - z) * n
    t1 = jnp.maximum(
        _layer_norm_tc(hcur @ wp1_ref[...] + bp1_ref[...], g1_ref[...], bn1_ref[...]), 0.0)
    t2 = jnp.maximum(
        _layer_norm_tc(t1 @ wp2_ref[...] + bp2_ref[...], g2_ref[...], bn2_ref[...]), 0.0)
    y_ref[...] = t2 @ wp3_ref[...] + bp3_ref[...]


_stage_c = pl.pallas_call(
    _stage_c_body,
    grid=(GRID,),
    in_specs=[
        pl.BlockSpec((NW, BLK), lambda i: (0, i)),
        pl.BlockSpec((NC, BLK, H), lambda i: (0, i, 0)),
        pl.BlockSpec((BLK, H), lambda i: (i, 0)),
        pl.BlockSpec((1, H), lambda i: (0, 0)),
        pl.BlockSpec((H, 3 * H), lambda i: (0, 0)),
        pl.BlockSpec((1, 3 * H), lambda i: (0, 0)),
        pl.BlockSpec((1, 3 * H), lambda i: (0, 0)),
        pl.BlockSpec((H, H), lambda i: (0, 0)),
        pl.BlockSpec((1, H), lambda i: (0, 0)),
        pl.BlockSpec((1, H), lambda i: (0, 0)),
        pl.BlockSpec((1, H), lambda i: (0, 0)),
        pl.BlockSpec((H, H), lambda i: (0, 0)),
        pl.BlockSpec((1, H), lambda i: (0, 0)),
        pl.BlockSpec((1, H), lambda i: (0, 0)),
        pl.BlockSpec((1, H), lambda i: (0, 0)),
        pl.BlockSpec((H, 1), lambda i: (0, 0)),
        pl.BlockSpec((1, 1), lambda i: (0, 0)),
    ],
    out_specs=[pl.BlockSpec((BLK, 1), lambda i: (i, 0))],
    out_shape=[jax.ShapeDtypeStruct((NP, 1), jnp.float32)],
)


def kernel(x, edge_index, W_lin, b_lin, Wc1, bc1, Wc2, bc2, W_ih, W_hh, b_ih, b_hh,
           Wp1, bp1, g1, bn1, Wp2, bp2, g2, bn2, Wp3, bp3):
    src = edge_index[0].reshape(NW, NCHUNKS, CHUNK)
    dst = edge_index[1].reshape(NW, NCHUNKS, CHUNK)
    dst_flat = edge_index[1].reshape(NW, EPW)
    xp = jnp.pad(x, ((0, NP - N), (0, 0)))
    zc = jnp.zeros((16, H), jnp.float32)

    degp = _deg_kernel()(dst_flat)
    hw1, g1m = _stage_a(degp, xp, W_lin, b_lin.reshape(1, H), Wc1)
    acc1 = _conv_kernel()(g1m, src, dst, zc)
    hw2, g2m = _stage_b(degp, acc1, hw1, bc1.reshape(1, H), Wc2)
    acc2 = _conv_kernel()(g2m, src, dst, zc)
    (y,) = _stage_c(
        degp, acc2, hw2, bc2.reshape(1, H), W_ih, b_ih.reshape(1, 3 * H),
        b_hh.reshape(1, 3 * H),
        Wp1, bp1.reshape(1, H), g1.reshape(1, H), bn1.reshape(1, H),
        Wp2, bp2.reshape(1, H), g2.reshape(1, H), bn2.reshape(1, H),
        Wp3, bp3.reshape(1, 1),
    )
    return y[:N]


# speedup vs baseline: 30.0493x; 1.2239x over previous
import functools

import jax
import jax.numpy as jnp
from jax import lax
from jax.experimental import pallas as pl
from jax.experimental.pallas import tpu as pltpu
from jax.experimental.pallas import tpu_sc as plsc

N = 10000
E = 320000
F = 128
H = 128

NP = 10240
NC = 2
NS = 16
NW = NC * NS
EPW = E // NW
CHUNK = 80
NCHUNKS = EPW // CHUNK
NBUF = 4
NGROUP = NCHUNKS // NBUF
RPT = NP // NS
BLK = 512
GRID = NP // BLK

@functools.cache
def _mesh():
    return plsc.VectorSubcoreMesh(
        core_axis_name="c", subcore_axis_name="s", num_cores=NC, num_subcores=NS
    )


def _deg_body(dst_hbm, out_hbm, deg_v, idx_v):
    c = lax.axis_index("c")
    s = lax.axis_index("s")
    wid = s * NC + c

    zeros16 = jnp.zeros((16,), jnp.float32)

    def zloop(i, carry):
        deg_v[pl.ds(i * 16, 16)] = zeros16
        return carry

    lax.fori_loop(0, NP // 16, zloop, 0)

    pltpu.sync_copy(dst_hbm.at[wid], idx_v)
    ones16 = jnp.ones((16,), jnp.float32)

    def body(i, carry):
        idx = idx_v[pl.ds(i * 16, 16)]
        plsc.addupdate_scatter(deg_v, [idx], ones16)
        return carry

    lax.fori_loop(0, EPW // 16, body, 0)
    pltpu.sync_copy(deg_v, out_hbm.at[wid])


@functools.cache
def _deg_kernel():
    return pl.kernel(
        _deg_body,
        out_type=jax.ShapeDtypeStruct((NW, NP), jnp.float32),
        mesh=_mesh(),
        scratch_types=[
            pltpu.VMEM((NP,), jnp.float32),
            pltpu.VMEM((EPW,), jnp.int32),
        ],
        compiler_params=pltpu.CompilerParams(needs_layout_passes=False),
    )


def _conv_body(g_hbm, src_hbm, dst_hbm, zc_hbm, out_hbm,
               acc_sh, src_v, dst_v, rows_v, zero_v, gsem, ssem, isem):
    c = lax.axis_index("c")
    s = lax.axis_index("s")
    wid = s * NC + c

    pltpu.sync_copy(zc_hbm, zero_v)

    def zacc(i, carry):
        pltpu.sync_copy(zero_v, acc_sh.at[pl.ds((i * NS + s) * 16, 16)])
        return carry

    lax.fori_loop(0, RPT // 16, zacc, 0)
    plsc.subcore_barrier()

    pltpu.sync_copy(src_hbm.at[wid, pl.ds(0, NBUF)], src_v.at[0])
    pltpu.sync_copy(dst_hbm.at[wid, pl.ds(0, NBUF)], dst_v.at[0])
    for b in range(NBUF):
        pltpu.async_copy(g_hbm.at[src_v.at[0, b]], rows_v.at[b], gsem)

    def group(gi, carry):
        p = lax.rem(gi, 2)
        pn = lax.rem(gi + 1, 2)
        notlast = gi + 1 < NGROUP
        jn = (gi + 1) * NBUF

        @pl.when(notlast)
        def _():
            pltpu.async_copy(src_hbm.at[wid, pl.ds(jn, NBUF)], src_v.at[pn], isem)
            pltpu.async_copy(dst_hbm.at[wid, pl.ds(jn, NBUF)], dst_v.at[pn], isem)

        for b in range(NBUF):
            pltpu.make_async_copy(
                g_hbm.at[src_v.at[p, b]], rows_v.at[b], gsem
            ).wait()
            pltpu.async_copy(
                rows_v.at[b], acc_sh.at[dst_v.at[p, b]], ssem, add=True
            )

        @pl.when(notlast)
        def _():
            pltpu.make_async_copy(
                src_hbm.at[wid, pl.ds(jn, NBUF)], src_v.at[pn], isem
            ).wait()
            pltpu.make_async_copy(
                dst_hbm.at[wid, pl.ds(jn, NBUF)], dst_v.at[pn], isem
            ).wait()

        for b in range(NBUF):
            pltpu.make_async_copy(
                rows_v.at[b], acc_sh.at[dst_v.at[p, b]], ssem
            ).wait()

            @pl.when(notlast)
            def _():
                pltpu.async_copy(g_hbm.at[src_v.at[pn, b]], rows_v.at[b], gsem)

        return carry

    lax.fori_loop(0, NGROUP, group, 0)

    for j in range(NGROUP * NBUF, NCHUNKS):
        pltpu.sync_copy(src_hbm.at[wid, pl.ds(j, 1)], src_v.at[0, pl.ds(0, 1)])
        pltpu.sync_copy(dst_hbm.at[wid, pl.ds(j, 1)], dst_v.at[0, pl.ds(0, 1)])
        pltpu.async_copy(g_hbm.at[src_v.at[0, 0]], rows_v.at[0], gsem).wait()
        pltpu.async_copy(
            rows_v.at[0], acc_sh.at[dst_v.at[0, 0]], ssem, add=True
        ).wait()
    plsc.subcore_barrier()

    row0 = s * RPT
    pltpu.sync_copy(acc_sh.at[pl.ds(row0, RPT)], out_hbm.at[c, pl.ds(row0, RPT)])


@functools.cache
def _conv_kernel():
    return pl.kernel(
        _conv_body,
        out_type=jax.ShapeDtypeStruct((NC, NP, H), jnp.float32),
        mesh=_mesh(),
        scratch_types=[
            pltpu.VMEM_SHARED((NP, H), jnp.float32),
            pltpu.VMEM((2, NBUF, CHUNK), jnp.int32),
            pltpu.VMEM((2, NBUF, CHUNK), jnp.int32),
            pltpu.VMEM((NBUF, CHUNK, H), jnp.float32),
            pltpu.VMEM((16, H), jnp.float32),
            pltpu.SemaphoreType.DMA,
            pltpu.SemaphoreType.DMA,
            pltpu.SemaphoreType.DMA,
        ],
        compiler_params=pltpu.CompilerParams(use_tc_tiling_on_sc=False),
    )


def _dinv_from_partials(degp):
    deg = jnp.sum(degp, axis=0) + 1.0
    return lax.rsqrt(deg)[:, None]


def _layer_norm_tc(t, g, b):
    mu = jnp.mean(t, axis=-1, keepdims=True)
    v = jnp.mean((t - mu) ** 2, axis=-1, keepdims=True)
    return (t - mu) / jnp.sqrt(v + 1e-5) * g + b


def _stage_a_body(degp_ref, x_ref, wlin_ref, blin_ref, wc1_ref, hw1_ref, g1_ref):
    dinv = _dinv_from_partials(degp_ref[...])
    h0 = jnp.maximum(x_ref[...] @ wlin_ref[...] + blin_ref[...], 0.0)
    hw1 = h0 @ wc1_ref[...]
    hw1_ref[...] = hw1
    g1_ref[...] = hw1 * dinv


_stage_a = pl.pallas_call(
    _stage_a_body,
    grid=(GRID,),
    in_specs=[
        pl.BlockSpec((NW, BLK), lambda i: (0, i)),
        pl.BlockSpec((BLK, F), lambda i: (i, 0)),
        pl.BlockSpec((F, H), lambda i: (0, 0)),
        pl.BlockSpec((1, H), lambda i: (0, 0)),
        pl.BlockSpec((H, H), lambda i: (0, 0)),
    ],
    out_specs=[
        pl.BlockSpec((BLK, H), lambda i: (i, 0)),
        pl.BlockSpec((BLK, H), lambda i: (i, 0)),
    ],
    out_shape=[
        jax.ShapeDtypeStruct((NP, H), jnp.float32),
        jax.ShapeDtypeStruct((NP, H), jnp.float32),
    ],
)


def _stage_b_body(degp_ref, acc_ref, hw1_ref, bc1_ref, wc2_ref, hw2_ref, g2_ref):
    dinv = _dinv_from_partials(degp_ref[...])
    a = acc_ref[...]
    conv = dinv * (a[0] + a[1]) + (dinv * dinv) * hw1_ref[...] + bc1_ref[...]
    h1 = jnp.maximum(conv, 0.0)
    hw2 = h1 @ wc2_ref[...]
    hw2_ref[...] = hw2
    g2_ref[...] = hw2 * dinv


_stage_b = pl.pallas_call(
    _stage_b_body,
    grid=(GRID,),
    in_specs=[
        pl.BlockSpec((NW, BLK), lambda i: (0, i)),
        pl.BlockSpec((NC, BLK, H), lambda i: (0, i, 0)),
        pl.BlockSpec((BLK, H), lambda i: (i, 0)),
        pl.BlockSpec((1, H), lambda i: (0, 0)),
        pl.BlockSpec((H, H), lambda i: (0, 0)),
    ],
    out_specs=[
        pl.BlockSpec((BLK, H), lambda i: (i, 0)),
        pl.BlockSpec((BLK, H), lambda i: (i, 0)),
    ],
    out_shape=[
        jax.ShapeDtypeStruct((NP, H), jnp.float32),
        jax.ShapeDtypeStruct((NP, H), jnp.float32),
    ],
)


def _stage_c_body(degp_ref, acc_ref, hw2_ref, bc2_ref, wih_ref, bih_ref, bhh_ref,
                  wp1_ref, bp1_ref, g1_ref, bn1_ref,
                  wp2_ref, bp2_ref, g2_ref, bn2_ref,
                  wp3_ref, bp3_ref, y_ref):
    dinv = _dinv_from_partials(degp_ref[...])
    a = acc_ref[...]
    conv = dinv * (a[0] + a[1]) + (dinv * dinv) * hw2_ref[...] + bc2_ref[...]
    h2 = jnp.maximum(conv, 0.0)
    gi = h2 @ wih_ref[...] + bih_ref[...]
    bhh = bhh_ref[...]
    r = jax.nn.sigmoid(gi[:, :H] + bhh[:, :H])
    z = jax.nn.sigmoid(gi[:, H:2 * H] + bhh[:, H:2 * H])
    n = jnp.tanh(gi[:, 2 * H:] + r * bhh[:, 2 * H:])
    hcur = (1.0 - z) * n
    t1 = jnp.maximum(
        _layer_norm_tc(hcur @ wp1_ref[...] + bp1_ref[...], g1_ref[...], bn1_ref[...]), 0.0)
    t2 = jnp.maximum(
        _layer_norm_tc(t1 @ wp2_ref[...] + bp2_ref[...], g2_ref[...], bn2_ref[...]), 0.0)
    y_ref[...] = t2 @ wp3_ref[...] + bp3_ref[...]


_stage_c = pl.pallas_call(
    _stage_c_body,
    grid=(GRID,),
    in_specs=[
        pl.BlockSpec((NW, BLK), lambda i: (0, i)),
        pl.BlockSpec((NC, BLK, H), lambda i: (0, i, 0)),
        pl.BlockSpec((BLK, H), lambda i: (i, 0)),
        pl.BlockSpec((1, H), lambda i: (0, 0)),
        pl.BlockSpec((H, 3 * H), lambda i: (0, 0)),
        pl.BlockSpec((1, 3 * H), lambda i: (0, 0)),
        pl.BlockSpec((1, 3 * H), lambda i: (0, 0)),
        pl.BlockSpec((H, H), lambda i: (0, 0)),
        pl.BlockSpec((1, H), lambda i: (0, 0)),
        pl.BlockSpec((1, H), lambda i: (0, 0)),
        pl.BlockSpec((1, H), lambda i: (0, 0)),
        pl.BlockSpec((H, H), lambda i: (0, 0)),
        pl.BlockSpec((1, H), lambda i: (0, 0)),
        pl.BlockSpec((1, H), lambda i: (0, 0)),
        pl.BlockSpec((1, H), lambda i: (0, 0)),
        pl.BlockSpec((H, 1), lambda i: (0, 0)),
        pl.BlockSpec((1, 1), lambda i: (0, 0)),
    ],
    out_specs=[pl.BlockSpec((BLK, 1), lambda i: (i, 0))],
    out_shape=[jax.ShapeDtypeStruct((NP, 1), jnp.float32)],
)


def kernel(x, edge_index, W_lin, b_lin, Wc1, bc1, Wc2, bc2, W_ih, W_hh, b_ih, b_hh,
           Wp1, bp1, g1, bn1, Wp2, bp2, g2, bn2, Wp3, bp3):
    src = edge_index[0].reshape(NW, NCHUNKS, CHUNK)
    dst = edge_index[1].reshape(NW, NCHUNKS, CHUNK)
    dst_flat = edge_index[1].reshape(NW, EPW)
    xp = jnp.pad(x, ((0, NP - N), (0, 0)))
    zc = jnp.zeros((16, H), jnp.float32)

    degp = _deg_kernel()(dst_flat)
    hw1, g1m = _stage_a(degp, xp, W_lin, b_lin.reshape(1, H), Wc1)
    acc1 = _conv_kernel()(g1m, src, dst, zc)
    hw2, g2m = _stage_b(degp, acc1, hw1, bc1.reshape(1, H), Wc2)
    acc2 = _conv_kernel()(g2m, src, dst, zc)
    (y,) = _stage_c(
        degp, acc2, hw2, bc2.reshape(1, H), W_ih, b_ih.reshape(1, 3 * H),
        b_hh.reshape(1, 3 * H),
        Wp1, bp1.reshape(1, H), g1.reshape(1, H), bn1.reshape(1, H),
        Wp2, bp2.reshape(1, H), g2.reshape(1, H), bn2.reshape(1, H),
        Wp3, bp3.reshape(1, 1),
    )
    return y[:N]


# speedup vs baseline: 30.4939x; 1.0148x over previous
import functools

import jax
import jax.numpy as jnp
from jax import lax
from jax.experimental import pallas as pl
from jax.experimental.pallas import tpu as pltpu
from jax.experimental.pallas import tpu_sc as plsc

N = 10000
E = 320000
F = 128
H = 128

NP = 10240
NC = 2
NS = 16
NW = NC * NS
EPW = E // NW
CHUNK = 80
NCHUNKS = EPW // CHUNK
NBUF = 4
NGROUP = NCHUNKS // NBUF
RPT = NP // NS
BLK = 512
GRID = NP // BLK

@functools.cache
def _mesh():
    return plsc.VectorSubcoreMesh(
        core_axis_name="c", subcore_axis_name="s", num_cores=NC, num_subcores=NS
    )


def _deg_body(dst_hbm, out_hbm, deg_v, idx_v):
    c = lax.axis_index("c")
    s = lax.axis_index("s")
    wid = s * NC + c

    zeros16 = jnp.zeros((16,), jnp.float32)

    def zloop(i, carry):
        deg_v[pl.ds(i * 16, 16)] = zeros16
        return carry

    lax.fori_loop(0, NP // 16, zloop, 0)

    pltpu.sync_copy(dst_hbm.at[wid], idx_v)
    ones16 = jnp.ones((16,), jnp.float32)

    def body(i, carry):
        idx = idx_v[pl.ds(i * 16, 16)]
        plsc.addupdate_scatter(deg_v, [idx], ones16)
        return carry

    lax.fori_loop(0, EPW // 16, body, 0)
    pltpu.sync_copy(deg_v, out_hbm.at[wid])


@functools.cache
def _deg_kernel():
    return pl.kernel(
        _deg_body,
        out_type=jax.ShapeDtypeStruct((NW, NP), jnp.float32),
        mesh=_mesh(),
        scratch_types=[
            pltpu.VMEM((NP,), jnp.float32),
            pltpu.VMEM((EPW,), jnp.int32),
        ],
        compiler_params=pltpu.CompilerParams(needs_layout_passes=False),
    )


def _conv_body(g_hbm, src_hbm, dst_hbm, out_hbm,
               acc_sh, src_v, dst_v, rows_v, zero_v, gsem, ssem, isem):
    c = lax.axis_index("c")
    s = lax.axis_index("s")
    wid = s * NC + c

    zeros16 = jnp.zeros((16,), jnp.float32)
    for r in range(16):
        for cc in range(H // 16):
            zero_v[r, pl.ds(cc * 16, 16)] = zeros16

    def zacc(i, carry):
        pltpu.sync_copy(zero_v, acc_sh.at[pl.ds((i * NS + s) * 16, 16)])
        return carry

    lax.fori_loop(0, RPT // 16, zacc, 0)
    plsc.subcore_barrier()

    pltpu.sync_copy(src_hbm.at[wid, pl.ds(0, NBUF)], src_v.at[0])
    pltpu.sync_copy(dst_hbm.at[wid, pl.ds(0, NBUF)], dst_v.at[0])
    for b in range(NBUF):
        pltpu.async_copy(g_hbm.at[src_v.at[0, b]], rows_v.at[b], gsem)

    def group(gi, carry):
        p = lax.rem(gi, 2)
        pn = lax.rem(gi + 1, 2)
        notlast = gi + 1 < NGROUP
        jn = (gi + 1) * NBUF

        @pl.when(notlast)
        def _():
            pltpu.async_copy(src_hbm.at[wid, pl.ds(jn, NBUF)], src_v.at[pn], isem)
            pltpu.async_copy(dst_hbm.at[wid, pl.ds(jn, NBUF)], dst_v.at[pn], isem)

        for b in range(NBUF):
            pltpu.make_async_copy(
                g_hbm.at[src_v.at[p, b]], rows_v.at[b], gsem
            ).wait()
            pltpu.async_copy(
                rows_v.at[b], acc_sh.at[dst_v.at[p, b]], ssem, add=True
            )

        @pl.when(notlast)
        def _():
            pltpu.make_async_copy(
                src_hbm.at[wid, pl.ds(jn, NBUF)], src_v.at[pn], isem
            ).wait()
            pltpu.make_async_copy(
                dst_hbm.at[wid, pl.ds(jn, NBUF)], dst_v.at[pn], isem
            ).wait()

        for b in range(NBUF):
            pltpu.make_async_copy(
                rows_v.at[b], acc_sh.at[dst_v.at[p, b]], ssem
            ).wait()

            @pl.when(notlast)
            def _():
                pltpu.async_copy(g_hbm.at[src_v.at[pn, b]], rows_v.at[b], gsem)

        return carry

    lax.fori_loop(0, NGROUP, group, 0)

    for j in range(NGROUP * NBUF, NCHUNKS):
        pltpu.sync_copy(src_hbm.at[wid, pl.ds(j, 1)], src_v.at[0, pl.ds(0, 1)])
        pltpu.sync_copy(dst_hbm.at[wid, pl.ds(j, 1)], dst_v.at[0, pl.ds(0, 1)])
        pltpu.async_copy(g_hbm.at[src_v.at[0, 0]], rows_v.at[0], gsem).wait()
        pltpu.async_copy(
            rows_v.at[0], acc_sh.at[dst_v.at[0, 0]], ssem, add=True
        ).wait()
    plsc.subcore_barrier()

    row0 = s * RPT
    pltpu.sync_copy(acc_sh.at[pl.ds(row0, RPT)], out_hbm.at[c, pl.ds(row0, RPT)])


@functools.cache
def _conv_kernel():
    return pl.kernel(
        _conv_body,
        out_type=jax.ShapeDtypeStruct((NC, NP, H), jnp.float32),
        mesh=_mesh(),
        scratch_types=[
            pltpu.VMEM_SHARED((NP, H), jnp.float32),
            pltpu.VMEM((2, NBUF, CHUNK), jnp.int32),
            pltpu.VMEM((2, NBUF, CHUNK), jnp.int32),
            pltpu.VMEM((NBUF, CHUNK, H), jnp.float32),
            pltpu.VMEM((16, H), jnp.float32),
            pltpu.SemaphoreType.DMA,
            pltpu.SemaphoreType.DMA,
            pltpu.SemaphoreType.DMA,
        ],
        compiler_params=pltpu.CompilerParams(use_tc_tiling_on_sc=False),
    )


def _dinv_from_partials(degp):
    deg = jnp.sum(degp, axis=0) + 1.0
    return lax.rsqrt(deg)[:, None]


def _layer_norm_tc(t, g, b):
    mu = jnp.mean(t, axis=-1, keepdims=True)
    v = jnp.mean((t - mu) ** 2, axis=-1, keepdims=True)
    return (t - mu) / jnp.sqrt(v + 1e-5) * g + b


def _stage_a_body(degp_ref, x_ref, wlin_ref, blin_ref, wc1_ref, g1_ref):
    dinv = _dinv_from_partials(degp_ref[...])
    h0 = jnp.maximum(x_ref[...] @ wlin_ref[...] + blin_ref[...], 0.0)
    g1_ref[...] = (h0 @ wc1_ref[...]) * dinv


_stage_a = pl.pallas_call(
    _stage_a_body,
    grid=(GRID,),
    in_specs=[
        pl.BlockSpec((NW, BLK), lambda i: (0, i)),
        pl.BlockSpec((BLK, F), lambda i: (i, 0)),
        pl.BlockSpec((F, H), lambda i: (0, 0)),
        pl.BlockSpec((1, H), lambda i: (0, 0)),
        pl.BlockSpec((H, H), lambda i: (0, 0)),
    ],
    out_specs=[
        pl.BlockSpec((BLK, H), lambda i: (i, 0)),
    ],
    out_shape=[
        jax.ShapeDtypeStruct((NP, H), jnp.float32),
    ],
)


def _stage_b_body(degp_ref, acc_ref, g1_ref, bc1_ref, wc2_ref, g2_ref):
    dinv = _dinv_from_partials(degp_ref[...])
    a = acc_ref[...]
    conv = dinv * (a[0] + a[1] + g1_ref[...]) + bc1_ref[...]
    h1 = jnp.maximum(conv, 0.0)
    g2_ref[...] = (h1 @ wc2_ref[...]) * dinv


_stage_b = pl.pallas_call(
    _stage_b_body,
    grid=(GRID,),
    in_specs=[
        pl.BlockSpec((NW, BLK), lambda i: (0, i)),
        pl.BlockSpec((NC, BLK, H), lambda i: (0, i, 0)),
        pl.BlockSpec((BLK, H), lambda i: (i, 0)),
        pl.BlockSpec((1, H), lambda i: (0, 0)),
        pl.BlockSpec((H, H), lambda i: (0, 0)),
    ],
    out_specs=[
        pl.BlockSpec((BLK, H), lambda i: (i, 0)),
    ],
    out_shape=[
        jax.ShapeDtypeStruct((NP, H), jnp.float32),
    ],
)


def _stage_c_body(degp_ref, acc_ref, g2m_ref, bc2_ref, wih_ref, bih_ref, bhh_ref,
                  wp1_ref, bp1_ref, g1_ref, bn1_ref,
                  wp2_ref, bp2_ref, g2_ref, bn2_ref,
                  wp3_ref, bp3_ref, y_ref):
    dinv = _dinv_from_partials(degp_ref[...])
    a = acc_ref[...]
    conv = dinv * (a[0] + a[1] + g2m_ref[...]) + bc2_ref[...]
    h2 = jnp.maximum(conv, 0.0)
    gi = h2 @ wih_ref[...] + bih_ref[...]
    bhh = bhh_ref[...]
    r = jax.nn.sigmoid(gi[:, :H] + bhh[:, :H])
    z = jax.nn.sigmoid(gi[:, H:2 * H] + bhh[:, H:2 * H])
    n = jnp.tanh(gi[:, 2 * H:] + r * bhh[:, 2 * H:])
    hcur = (1.0 - z) * n
    t1 = jnp.maximum(
        _layer_norm_tc(hcur @ wp1_ref[...] + bp1_ref[...], g1_ref[...], bn1_ref[...]), 0.0)
    t2 = jnp.maximum(
        _layer_norm_tc(t1 @ wp2_ref[...] + bp2_ref[...], g2_ref[...], bn2_ref[...]), 0.0)
    y_ref[...] = t2 @ wp3_ref[...] + bp3_ref[...]


_stage_c = pl.pallas_call(
    _stage_c_body,
    grid=(GRID,),
    in_specs=[
        pl.BlockSpec((NW, BLK), lambda i: (0, i)),
        pl.BlockSpec((NC, BLK, H), lambda i: (0, i, 0)),
        pl.BlockSpec((BLK, H), lambda i: (i, 0)),
        pl.BlockSpec((1, H), lambda i: (0, 0)),
        pl.BlockSpec((H, 3 * H), lambda i: (0, 0)),
        pl.BlockSpec((1, 3 * H), lambda i: (0, 0)),
        pl.BlockSpec((1, 3 * H), lambda i: (0, 0)),
        pl.BlockSpec((H, H), lambda i: (0, 0)),
        pl.BlockSpec((1, H), lambda i: (0, 0)),
        pl.BlockSpec((1, H), lambda i: (0, 0)),
        pl.BlockSpec((1, H), lambda i: (0, 0)),
        pl.BlockSpec((H, H), lambda i: (0, 0)),
        pl.BlockSpec((1, H), lambda i: (0, 0)),
        pl.BlockSpec((1, H), lambda i: (0, 0)),
        pl.BlockSpec((1, H), lambda i: (0, 0)),
        pl.BlockSpec((H, 1), lambda i: (0, 0)),
        pl.BlockSpec((1, 1), lambda i: (0, 0)),
    ],
    out_specs=[pl.BlockSpec((BLK, 1), lambda i: (i, 0))],
    out_shape=[jax.ShapeDtypeStruct((NP, 1), jnp.float32)],
)


def kernel(x, edge_index, W_lin, b_lin, Wc1, bc1, Wc2, bc2, W_ih, W_hh, b_ih, b_hh,
           Wp1, bp1, g1, bn1, Wp2, bp2, g2, bn2, Wp3, bp3):
    src = edge_index[0].reshape(NW, NCHUNKS, CHUNK)
    dst = edge_index[1].reshape(NW, NCHUNKS, CHUNK)
    dst_flat = edge_index[1].reshape(NW, EPW)
    xp = jnp.pad(x, ((0, NP - N), (0, 0)))

    degp = _deg_kernel()(dst_flat)
    (g1m,) = _stage_a(degp, xp, W_lin, b_lin.reshape(1, H), Wc1)
    acc1 = _conv_kernel()(g1m, src, dst)
    (g2m,) = _stage_b(degp, acc1, g1m, bc1.reshape(1, H), Wc2)
    acc2 = _conv_kernel()(g2m, src, dst)
    (y,) = _stage_c(
        degp, acc2, g2m, bc2.reshape(1, H), W_ih, b_ih.reshape(1, 3 * H),
        b_hh.reshape(1, 3 * H),
        Wp1, bp1.reshape(1, H), g1.reshape(1, H), bn1.reshape(1, H),
        Wp2, bp2.reshape(1, H), g2.reshape(1, H), bn2.reshape(1, H),
        Wp3, bp3.reshape(1, 1),
    )
    return y[:N]


# speedup vs baseline: 31.2857x; 1.0260x over previous
import functools

import jax
import jax.numpy as jnp
from jax import lax
from jax.experimental import pallas as pl
from jax.experimental.pallas import tpu as pltpu
from jax.experimental.pallas import tpu_sc as plsc

N = 10000
E = 320000
F = 128
H = 128

NP = 10240
NC = 2
NS = 16
NW = NC * NS
EPW = E // NW
CHUNK = 80
NCHUNKS = EPW // CHUNK
NBUF = 4
NGROUP = NCHUNKS // NBUF
RPT = NP // NS
BLK = 1000
GRID = N // BLK

@functools.cache
def _mesh():
    return plsc.VectorSubcoreMesh(
        core_axis_name="c", subcore_axis_name="s", num_cores=NC, num_subcores=NS
    )


def _deg_body(dst_hbm, out_hbm, deg_v, idx_v):
    c = lax.axis_index("c")
    s = lax.axis_index("s")
    wid = s * NC + c

    zeros16 = jnp.zeros((16,), jnp.float32)

    def zloop(i, carry):
        deg_v[pl.ds(i * 16, 16)] = zeros16
        return carry

    lax.fori_loop(0, NP // 16, zloop, 0)

    pltpu.sync_copy(dst_hbm.at[wid], idx_v)
    ones16 = jnp.ones((16,), jnp.float32)

    def body(i, carry):
        jc = i // (CHUNK // 16)
        k = lax.rem(i, CHUNK // 16)
        idx = idx_v[jc, pl.ds(k * 16, 16)]
        plsc.addupdate_scatter(deg_v, [idx], ones16)
        return carry

    lax.fori_loop(0, EPW // 16, body, 0)
    pltpu.sync_copy(deg_v, out_hbm.at[wid])


@functools.cache
def _deg_kernel():
    return pl.kernel(
        _deg_body,
        out_type=jax.ShapeDtypeStruct((NW, NP), jnp.float32),
        mesh=_mesh(),
        scratch_types=[
            pltpu.VMEM((NP,), jnp.float32),
            pltpu.VMEM((NCHUNKS, CHUNK), jnp.int32),
        ],
        compiler_params=pltpu.CompilerParams(
            needs_layout_passes=False, use_tc_tiling_on_sc=False
        ),
    )


def _conv_body(g_hbm, src_hbm, dst_hbm, out_hbm,
               acc_sh, src_v, dst_v, rows_v, zero_v, gsem, ssem, isem):
    c = lax.axis_index("c")
    s = lax.axis_index("s")
    wid = s * NC + c

    zeros16 = jnp.zeros((16,), jnp.float32)
    for r in range(16):
        for cc in range(H // 16):
            zero_v[r, pl.ds(cc * 16, 16)] = zeros16

    def zacc(i, carry):
        pltpu.sync_copy(zero_v, acc_sh.at[pl.ds((i * NS + s) * 16, 16)])
        return carry

    lax.fori_loop(0, RPT // 16, zacc, 0)
    plsc.subcore_barrier()

    pltpu.sync_copy(src_hbm.at[wid, pl.ds(0, NBUF)], src_v.at[0])
    pltpu.sync_copy(dst_hbm.at[wid, pl.ds(0, NBUF)], dst_v.at[0])
    for b in range(NBUF):
        pltpu.async_copy(g_hbm.at[src_v.at[0, b]], rows_v.at[b], gsem)

    def group(gi, carry):
        p = lax.rem(gi, 2)
        pn = lax.rem(gi + 1, 2)
        notlast = gi + 1 < NGROUP
        jn = (gi + 1) * NBUF

        @pl.when(notlast)
        def _():
            pltpu.async_copy(src_hbm.at[wid, pl.ds(jn, NBUF)], src_v.at[pn], isem)
            pltpu.async_copy(dst_hbm.at[wid, pl.ds(jn, NBUF)], dst_v.at[pn], isem)

        for b in range(NBUF):
            pltpu.make_async_copy(
                g_hbm.at[src_v.at[p, b]], rows_v.at[b], gsem
            ).wait()
            pltpu.async_copy(
                rows_v.at[b], acc_sh.at[dst_v.at[p, b]], ssem, add=True
            )

        @pl.when(notlast)
        def _():
            pltpu.make_async_copy(
                src_hbm.at[wid, pl.ds(jn, NBUF)], src_v.at[pn], isem
            ).wait()
            pltpu.make_async_copy(
                dst_hbm.at[wid, pl.ds(jn, NBUF)], dst_v.at[pn], isem
            ).wait()

        for b in range(NBUF):
            pltpu.make_async_copy(
                rows_v.at[b], acc_sh.at[dst_v.at[p, b]], ssem
            ).wait()

            @pl.when(notlast)
            def _():
                pltpu.async_copy(g_hbm.at[src_v.at[pn, b]], rows_v.at[b], gsem)

        return carry

    lax.fori_loop(0, NGROUP, group, 0)

    for j in range(NGROUP * NBUF, NCHUNKS):
        pltpu.sync_copy(src_hbm.at[wid, pl.ds(j, 1)], src_v.at[0, pl.ds(0, 1)])
        pltpu.sync_copy(dst_hbm.at[wid, pl.ds(j, 1)], dst_v.at[0, pl.ds(0, 1)])
        pltpu.async_copy(g_hbm.at[src_v.at[0, 0]], rows_v.at[0], gsem).wait()
        pltpu.async_copy(
            rows_v.at[0], acc_sh.at[dst_v.at[0, 0]], ssem, add=True
        ).wait()
    plsc.subcore_barrier()

    row0 = s * RPT
    pltpu.sync_copy(acc_sh.at[pl.ds(row0, RPT)], out_hbm.at[c, pl.ds(row0, RPT)])


@functools.cache
def _conv_kernel():
    return pl.kernel(
        _conv_body,
        out_type=jax.ShapeDtypeStruct((NC, NP, H), jnp.float32),
        mesh=_mesh(),
        scratch_types=[
            pltpu.VMEM_SHARED((NP, H), jnp.float32),
            pltpu.VMEM((2, NBUF, CHUNK), jnp.int32),
            pltpu.VMEM((2, NBUF, CHUNK), jnp.int32),
            pltpu.VMEM((NBUF, CHUNK, H), jnp.float32),
            pltpu.VMEM((16, H), jnp.float32),
            pltpu.SemaphoreType.DMA,
            pltpu.SemaphoreType.DMA,
            pltpu.SemaphoreType.DMA,
        ],
        compiler_params=pltpu.CompilerParams(use_tc_tiling_on_sc=False),
    )


def _dinv_body(degp_ref, dinv_ref):
    deg = jnp.sum(degp_ref[...], axis=0) + 1.0
    dinv_ref[...] = lax.rsqrt(deg)[:, None]


_dinv_kernel = pl.pallas_call(
    _dinv_body,
    grid=(NP // 1024,),
    in_specs=[pl.BlockSpec((NW, 1024), lambda i: (0, i))],
    out_specs=[pl.BlockSpec((1024, 1), lambda i: (i, 0))],
    out_shape=[jax.ShapeDtypeStruct((NP, 1), jnp.float32)],
)


def _layer_norm_tc(t, g, b):
    mu = jnp.mean(t, axis=-1, keepdims=True)
    v = jnp.mean((t - mu) ** 2, axis=-1, keepdims=True)
    return (t - mu) / jnp.sqrt(v + 1e-5) * g + b


def _stage_a_body(dinv_ref, x_ref, wlin_ref, blin_ref, wc1_ref, g1_ref):
    dinv = dinv_ref[...]
    h0 = jnp.maximum(x_ref[...] @ wlin_ref[...] + blin_ref[...], 0.0)
    g1_ref[...] = (h0 @ wc1_ref[...]) * dinv


_stage_a = pl.pallas_call(
    _stage_a_body,
    grid=(GRID,),
    in_specs=[
        pl.BlockSpec((BLK, 1), lambda i: (i, 0)),
        pl.BlockSpec((BLK, F), lambda i: (i, 0)),
        pl.BlockSpec((F, H), lambda i: (0, 0)),
        pl.BlockSpec((1, H), lambda i: (0, 0)),
        pl.BlockSpec((H, H), lambda i: (0, 0)),
    ],
    out_specs=[
        pl.BlockSpec((BLK, H), lambda i: (i, 0)),
    ],
    out_shape=[
        jax.ShapeDtypeStruct((N, H), jnp.float32),
    ],
)


def _stage_b_body(dinv_ref, acc_ref, g1_ref, bc1_ref, wc2_ref, g2_ref):
    dinv = dinv_ref[...]
    a = acc_ref[...]
    conv = dinv * (a[0] + a[1] + g1_ref[...]) + bc1_ref[...]
    h1 = jnp.maximum(conv, 0.0)
    g2_ref[...] = (h1 @ wc2_ref[...]) * dinv


_stage_b = pl.pallas_call(
    _stage_b_body,
    grid=(GRID,),
    in_specs=[
        pl.BlockSpec((BLK, 1), lambda i: (i, 0)),
        pl.BlockSpec((NC, BLK, H), lambda i: (0, i, 0)),
        pl.BlockSpec((BLK, H), lambda i: (i, 0)),
        pl.BlockSpec((1, H), lambda i: (0, 0)),
        pl.BlockSpec((H, H), lambda i: (0, 0)),
    ],
    out_specs=[
        pl.BlockSpec((BLK, H), lambda i: (i, 0)),
    ],
    out_shape=[
        jax.ShapeDtypeStruct((N, H), jnp.float32),
    ],
)


def _stage_c_body(dinv_ref, acc_ref, g2m_ref, bc2_ref, wih_ref, bih_ref, bhh_ref,
                  wp1_ref, bp1_ref, g1_ref, bn1_ref,
                  wp2_ref, bp2_ref, g2_ref, bn2_ref,
                  wp3_ref, bp3_ref, y_ref):
    dinv = dinv_ref[...]
    a = acc_ref[...]
    conv = dinv * (a[0] + a[1] + g2m_ref[...]) + bc2_ref[...]
    h2 = jnp.maximum(conv, 0.0)
    gi = h2 @ wih_ref[...] + bih_ref[...]
    bhh = bhh_ref[...]
    r = jax.nn.sigmoid(gi[:, :H] + bhh[:, :H])
    z = jax.nn.sigmoid(gi[:, H:2 * H] + bhh[:, H:2 * H])
    n = jnp.tanh(gi[:, 2 * H:] + r * bhh[:, 2 * H:])
    hcur = (1.0 - z) * n
    t1 = jnp.maximum(
        _layer_norm_tc(hcur @ wp1_ref[...] + bp1_ref[...], g1_ref[...], bn1_ref[...]), 0.0)
    t2 = jnp.maximum(
        _layer_norm_tc(t1 @ wp2_ref[...] + bp2_ref[...], g2_ref[...], bn2_ref[...]), 0.0)
    y_ref[...] = t2 @ wp3_ref[...] + bp3_ref[...]


_stage_c = pl.pallas_call(
    _stage_c_body,
    grid=(GRID,),
    in_specs=[
        pl.BlockSpec((BLK, 1), lambda i: (i, 0)),
        pl.BlockSpec((NC, BLK, H), lambda i: (0, i, 0)),
        pl.BlockSpec((BLK, H), lambda i: (i, 0)),
        pl.BlockSpec((1, H), lambda i: (0, 0)),
        pl.BlockSpec((H, 3 * H), lambda i: (0, 0)),
        pl.BlockSpec((1, 3 * H), lambda i: (0, 0)),
        pl.BlockSpec((1, 3 * H), lambda i: (0, 0)),
        pl.BlockSpec((H, H), lambda i: (0, 0)),
        pl.BlockSpec((1, H), lambda i: (0, 0)),
        pl.BlockSpec((1, H), lambda i: (0, 0)),
        pl.BlockSpec((1, H), lambda i: (0, 0)),
        pl.BlockSpec((H, H), lambda i: (0, 0)),
        pl.BlockSpec((1, H), lambda i: (0, 0)),
        pl.BlockSpec((1, H), lambda i: (0, 0)),
        pl.BlockSpec((1, H), lambda i: (0, 0)),
        pl.BlockSpec((H, 1), lambda i: (0, 0)),
        pl.BlockSpec((1, 1), lambda i: (0, 0)),
    ],
    out_specs=[pl.BlockSpec((BLK, 1), lambda i: (i, 0))],
    out_shape=[jax.ShapeDtypeStruct((N, 1), jnp.float32)],
)


def kernel(x, edge_index, W_lin, b_lin, Wc1, bc1, Wc2, bc2, W_ih, W_hh, b_ih, b_hh,
           Wp1, bp1, g1, bn1, Wp2, bp2, g2, bn2, Wp3, bp3):
    src = edge_index[0].reshape(NW, NCHUNKS, CHUNK)
    dst = edge_index[1].reshape(NW, NCHUNKS, CHUNK)

    degp = _deg_kernel()(dst)
    (dinv,) = _dinv_kernel(degp)
    (g1m,) = _stage_a(dinv, x, W_lin, b_lin.reshape(1, H), Wc1)
    acc1 = _conv_kernel()(g1m, src, dst)
    (g2m,) = _stage_b(dinv, acc1, g1m, bc1.reshape(1, H), Wc2)
    acc2 = _conv_kernel()(g2m, src, dst)
    (y,) = _stage_c(
        dinv, acc2, g2m, bc2.reshape(1, H), W_ih, b_ih.reshape(1, 3 * H),
        b_hh.reshape(1, 3 * H),
        Wp1, bp1.reshape(1, H), g1.reshape(1, H), bn1.reshape(1, H),
        Wp2, bp2.reshape(1, H), g2.reshape(1, H), bn2.reshape(1, H),
        Wp3, bp3.reshape(1, 1),
    )
    return y


# speedup vs baseline: 32.4327x; 1.0367x over previous
import functools

import jax
import jax.numpy as jnp
from jax import lax
from jax.experimental import pallas as pl
from jax.experimental.pallas import tpu as pltpu
from jax.experimental.pallas import tpu_sc as plsc

N = 10000
E = 320000
F = 128
H = 128

NP = 10240
NC = 2
NS = 16
NW = NC * NS
EPW = E // NW
CHUNK = 80
NCHUNKS = EPW // CHUNK
NBUF = 4
NGROUP = NCHUNKS // NBUF
RPT = NP // NS
BLK = 1000
GRID = N // BLK

@functools.cache
def _mesh():
    return plsc.VectorSubcoreMesh(
        core_axis_name="c", subcore_axis_name="s", num_cores=NC, num_subcores=NS
    )


def _deg_body(ei_hbm, out_hbm, deg_v, idx_v):
    c = lax.axis_index("c")
    s = lax.axis_index("s")
    wid = s * NC + c

    zeros16 = jnp.zeros((16,), jnp.float32)

    def zloop(i, carry):
        deg_v[pl.ds(i * 16, 16)] = zeros16
        return carry

    lax.fori_loop(0, NP // 16, zloop, 0)

    pltpu.sync_copy(ei_hbm.at[1, wid], idx_v)
    ones16 = jnp.ones((16,), jnp.float32)

    def body(i, carry):
        jc = i // (CHUNK // 16)
        k = lax.rem(i, CHUNK // 16)
        idx = idx_v[jc, pl.ds(k * 16, 16)]
        plsc.addupdate_scatter(deg_v, [idx], ones16)
        return carry

    lax.fori_loop(0, EPW // 16, body, 0)
    pltpu.sync_copy(deg_v, out_hbm.at[wid])


@functools.cache
def _deg_kernel():
    return pl.kernel(
        _deg_body,
        out_type=jax.ShapeDtypeStruct((NW, NP), jnp.float32),
        mesh=_mesh(),
        scratch_types=[
            pltpu.VMEM((NP,), jnp.float32),
            pltpu.VMEM((NCHUNKS, CHUNK), jnp.int32),
        ],
        compiler_params=pltpu.CompilerParams(
            needs_layout_passes=False, use_tc_tiling_on_sc=False
        ),
    )


def _conv_body(g_hbm, ei_hbm, out_hbm,
               acc_sh, src_v, dst_v, rows_v, zero_v, gsem, ssem, isem):
    c = lax.axis_index("c")
    s = lax.axis_index("s")
    wid = s * NC + c

    zeros16 = jnp.zeros((16,), jnp.float32)
    for r in range(16):
        for cc in range(H // 16):
            zero_v[r, pl.ds(cc * 16, 16)] = zeros16

    def zacc(i, carry):
        pltpu.sync_copy(zero_v, acc_sh.at[pl.ds((i * NS + s) * 16, 16)])
        return carry

    lax.fori_loop(0, RPT // 16, zacc, 0)
    plsc.subcore_barrier()

    pltpu.sync_copy(ei_hbm.at[0, wid, pl.ds(0, NBUF)], src_v.at[0])
    pltpu.sync_copy(ei_hbm.at[1, wid, pl.ds(0, NBUF)], dst_v.at[0])
    for b in range(NBUF):
        pltpu.async_copy(g_hbm.at[src_v.at[0, b]], rows_v.at[b], gsem)

    def group(gi, carry):
        p = lax.rem(gi, 2)
        pn = lax.rem(gi + 1, 2)
        notlast = gi + 1 < NGROUP
        jn = (gi + 1) * NBUF

        @pl.when(notlast)
        def _():
            pltpu.async_copy(ei_hbm.at[0, wid, pl.ds(jn, NBUF)], src_v.at[pn], isem)
            pltpu.async_copy(ei_hbm.at[1, wid, pl.ds(jn, NBUF)], dst_v.at[pn], isem)

        for b in range(NBUF):
            pltpu.make_async_copy(
                g_hbm.at[src_v.at[p, b]], rows_v.at[b], gsem
            ).wait()
            pltpu.async_copy(
                rows_v.at[b], acc_sh.at[dst_v.at[p, b]], ssem, add=True
            )

        @pl.when(notlast)
        def _():
            pltpu.make_async_copy(
                ei_hbm.at[0, wid, pl.ds(jn, NBUF)], src_v.at[pn], isem
            ).wait()
            pltpu.make_async_copy(
                ei_hbm.at[1, wid, pl.ds(jn, NBUF)], dst_v.at[pn], isem
            ).wait()

        for b in range(NBUF):
            pltpu.make_async_copy(
                rows_v.at[b], acc_sh.at[dst_v.at[p, b]], ssem
            ).wait()

            @pl.when(notlast)
            def _():
                pltpu.async_copy(g_hbm.at[src_v.at[pn, b]], rows_v.at[b], gsem)

        return carry

    lax.fori_loop(0, NGROUP, group, 0)

    for j in range(NGROUP * NBUF, NCHUNKS):
        pltpu.sync_copy(ei_hbm.at[0, wid, pl.ds(j, 1)], src_v.at[0, pl.ds(0, 1)])
        pltpu.sync_copy(ei_hbm.at[1, wid, pl.ds(j, 1)], dst_v.at[0, pl.ds(0, 1)])
        pltpu.async_copy(g_hbm.at[src_v.at[0, 0]], rows_v.at[0], gsem).wait()
        pltpu.async_copy(
            rows_v.at[0], acc_sh.at[dst_v.at[0, 0]], ssem, add=True
        ).wait()
    plsc.subcore_barrier()

    row0 = s * RPT
    pltpu.sync_copy(acc_sh.at[pl.ds(row0, RPT)], out_hbm.at[c, pl.ds(row0, RPT)])


@functools.cache
def _conv_kernel():
    return pl.kernel(
        _conv_body,
        out_type=jax.ShapeDtypeStruct((NC, NP, H), jnp.float32),
        mesh=_mesh(),
        scratch_types=[
            pltpu.VMEM_SHARED((NP, H), jnp.float32),
            pltpu.VMEM((2, NBUF, CHUNK), jnp.int32),
            pltpu.VMEM((2, NBUF, CHUNK), jnp.int32),
            pltpu.VMEM((NBUF, CHUNK, H), jnp.float32),
            pltpu.VMEM((16, H), jnp.float32),
            pltpu.SemaphoreType.DMA,
            pltpu.SemaphoreType.DMA,
            pltpu.SemaphoreType.DMA,
        ],
        compiler_params=pltpu.CompilerParams(use_tc_tiling_on_sc=False),
    )


def _dinv_body(degp_ref, dinv_ref):
    deg = jnp.sum(degp_ref[...], axis=0) + 1.0
    dinv_ref[...] = lax.rsqrt(deg)[:, None]


_dinv_kernel = pl.pallas_call(
    _dinv_body,
    grid=(NP // 1024,),
    in_specs=[pl.BlockSpec((NW, 1024), lambda i: (0, i))],
    out_specs=[pl.BlockSpec((1024, 1), lambda i: (i, 0))],
    out_shape=[jax.ShapeDtypeStruct((NP, 1), jnp.float32)],
)


def _layer_norm_tc(t, g, b):
    mu = jnp.mean(t, axis=-1, keepdims=True)
    v = jnp.mean((t - mu) ** 2, axis=-1, keepdims=True)
    return (t - mu) / jnp.sqrt(v + 1e-5) * g + b


def _stage_a_body(dinv_ref, x_ref, wlin_ref, blin_ref, wc1_ref, g1_ref):
    dinv = dinv_ref[...]
    h0 = jnp.maximum(x_ref[...] @ wlin_ref[...] + blin_ref[...], 0.0)
    g1_ref[...] = (h0 @ wc1_ref[...]) * dinv


_stage_a = pl.pallas_call(
    _stage_a_body,
    grid=(GRID,),
    in_specs=[
        pl.BlockSpec((BLK, 1), lambda i: (i, 0)),
        pl.BlockSpec((BLK, F), lambda i: (i, 0)),
        pl.BlockSpec((F, H), lambda i: (0, 0)),
        pl.BlockSpec((1, H), lambda i: (0, 0)),
        pl.BlockSpec((H, H), lambda i: (0, 0)),
    ],
    out_specs=[
        pl.BlockSpec((BLK, H), lambda i: (i, 0)),
    ],
    out_shape=[
        jax.ShapeDtypeStruct((N, H), jnp.float32),
    ],
)


def _stage_b_body(dinv_ref, acc_ref, g1_ref, bc1_ref, wc2_ref, g2_ref):
    dinv = dinv_ref[...]
    a = acc_ref[...]
    conv = dinv * (a[0] + a[1] + g1_ref[...]) + bc1_ref[...]
    h1 = jnp.maximum(conv, 0.0)
    g2_ref[...] = (h1 @ wc2_ref[...]) * dinv


_stage_b = pl.pallas_call(
    _stage_b_body,
    grid=(GRID,),
    in_specs=[
        pl.BlockSpec((BLK, 1), lambda i: (i, 0)),
        pl.BlockSpec((NC, BLK, H), lambda i: (0, i, 0)),
        pl.BlockSpec((BLK, H), lambda i: (i, 0)),
        pl.BlockSpec((1, H), lambda i: (0, 0)),
        pl.BlockSpec((H, H), lambda i: (0, 0)),
    ],
    out_specs=[
        pl.BlockSpec((BLK, H), lambda i: (i, 0)),
    ],
    out_shape=[
        jax.ShapeDtypeStruct((N, H), jnp.float32),
    ],
)


def _stage_c_body(dinv_ref, acc_ref, g2m_ref, bc2_ref, wih_ref, bih_ref, bhh_ref,
                  wp1_ref, bp1_ref, g1_ref, bn1_ref,
                  wp2_ref, bp2_ref, g2_ref, bn2_ref,
                  wp3_ref, bp3_ref, y_ref):
    dinv = dinv_ref[...]
    a = acc_ref[...]
    conv = dinv * (a[0] + a[1] + g2m_ref[...]) + bc2_ref[...]
    h2 = jnp.maximum(conv, 0.0)
    gi = h2 @ wih_ref[...] + bih_ref[...]
    bhh = bhh_ref[...]
    r = jax.nn.sigmoid(gi[:, :H] + bhh[:, :H])
    z = jax.nn.sigmoid(gi[:, H:2 * H] + bhh[:, H:2 * H])
    n = jnp.tanh(gi[:, 2 * H:] + r * bhh[:, 2 * H:])
    hcur = (1.0 - z) * n
    t1 = jnp.maximum(
        _layer_norm_tc(hcur @ wp1_ref[...] + bp1_ref[...], g1_ref[...], bn1_ref[...]), 0.0)
    t2 = jnp.maximum(
        _layer_norm_tc(t1 @ wp2_ref[...] + bp2_ref[...], g2_ref[...], bn2_ref[...]), 0.0)
    y_ref[...] = t2 @ wp3_ref[...] + bp3_ref[...]


_stage_c = pl.pallas_call(
    _stage_c_body,
    grid=(GRID,),
    in_specs=[
        pl.BlockSpec((BLK, 1), lambda i: (i, 0)),
        pl.BlockSpec((NC, BLK, H), lambda i: (0, i, 0)),
        pl.BlockSpec((BLK, H), lambda i: (i, 0)),
        pl.BlockSpec((1, H), lambda i: (0, 0)),
        pl.BlockSpec((H, 3 * H), lambda i: (0, 0)),
        pl.BlockSpec((1, 3 * H), lambda i: (0, 0)),
        pl.BlockSpec((1, 3 * H), lambda i: (0, 0)),
        pl.BlockSpec((H, H), lambda i: (0, 0)),
        pl.BlockSpec((1, H), lambda i: (0, 0)),
        pl.BlockSpec((1, H), lambda i: (0, 0)),
        pl.BlockSpec((1, H), lambda i: (0, 0)),
        pl.BlockSpec((H, H), lambda i: (0, 0)),
        pl.BlockSpec((1, H), lambda i: (0, 0)),
        pl.BlockSpec((1, H), lambda i: (0, 0)),
        pl.BlockSpec((1, H), lambda i: (0, 0)),
        pl.BlockSpec((H, 1), lambda i: (0, 0)),
        pl.BlockSpec((1, 1), lambda i: (0, 0)),
    ],
    out_specs=[pl.BlockSpec((BLK, 1), lambda i: (i, 0))],
    out_shape=[jax.ShapeDtypeStruct((N, 1), jnp.float32)],
)


def kernel(x, edge_index, W_lin, b_lin, Wc1, bc1, Wc2, bc2, W_ih, W_hh, b_ih, b_hh,
           Wp1, bp1, g1, bn1, Wp2, bp2, g2, bn2, Wp3, bp3):
    ei = edge_index.reshape(2, NW, NCHUNKS, CHUNK)

    degp = _deg_kernel()(ei)
    (dinv,) = _dinv_kernel(degp)
    (g1m,) = _stage_a(dinv, x, W_lin, b_lin.reshape(1, H), Wc1)
    acc1 = _conv_kernel()(g1m, ei)
    (g2m,) = _stage_b(dinv, acc1, g1m, bc1.reshape(1, H), Wc2)
    acc2 = _conv_kernel()(g2m, ei)
    (y,) = _stage_c(
        dinv, acc2, g2m, bc2.reshape(1, H), W_ih, b_ih.reshape(1, 3 * H),
        b_hh.reshape(1, 3 * H),
        Wp1, bp1.reshape(1, H), g1.reshape(1, H), bn1.reshape(1, H),
        Wp2, bp2.reshape(1, H), g2.reshape(1, H), bn2.reshape(1, H),
        Wp3, bp3.reshape(1, 1),
    )
    return y


# speedup vs baseline: 32.4493x; 1.0005x over previous
import functools

import jax
import jax.numpy as jnp
from jax import lax
from jax.experimental import pallas as pl
from jax.experimental.pallas import tpu as pltpu
from jax.experimental.pallas import tpu_sc as plsc

N = 10000
E = 320000
F = 128
H = 128

NP = 10240
NC = 2
NS = 16
NW = NC * NS
EPW = E // NW
CHUNK = 80
NCHUNKS = EPW // CHUNK
NBUF = 4
NGROUP = NCHUNKS // NBUF
RPT = NP // NS
BLK = 1000
GRID = N // BLK

@functools.cache
def _mesh():
    return plsc.VectorSubcoreMesh(
        core_axis_name="c", subcore_axis_name="s", num_cores=NC, num_subcores=NS
    )


def _deg_body(ei_hbm, out_hbm, deg_v, idx_v):
    c = lax.axis_index("c")
    s = lax.axis_index("s")
    wid = s * NC + c

    zeros16 = jnp.zeros((16,), jnp.float32)

    def zloop(i, carry):
        deg_v[pl.ds(i * 16, 16)] = zeros16
        return carry

    lax.fori_loop(0, NP // 16, zloop, 0)

    pltpu.sync_copy(ei_hbm.at[1, wid], idx_v)
    ones16 = jnp.ones((16,), jnp.float32)

    def body(i, carry):
        jc = i // (CHUNK // 16)
        k = lax.rem(i, CHUNK // 16)
        idx = idx_v[jc, pl.ds(k * 16, 16)]
        plsc.addupdate_scatter(deg_v, [idx], ones16)
        return carry

    lax.fori_loop(0, EPW // 16, body, 0)
    pltpu.sync_copy(deg_v, out_hbm.at[wid])


@functools.cache
def _deg_kernel():
    return pl.kernel(
        _deg_body,
        out_type=jax.ShapeDtypeStruct((NW, NP), jnp.float32),
        mesh=_mesh(),
        scratch_types=[
            pltpu.VMEM((NP,), jnp.float32),
            pltpu.VMEM((NCHUNKS, CHUNK), jnp.int32),
        ],
        compiler_params=pltpu.CompilerParams(
            needs_layout_passes=False, use_tc_tiling_on_sc=False,
            skip_device_barrier=True,
        ),
    )


def _conv_body(g_hbm, ei_hbm, out_hbm,
               acc_sh, src_v, dst_v, rows_v, zero_v, gsem, ssem, isem):
    c = lax.axis_index("c")
    s = lax.axis_index("s")
    wid = s * NC + c

    zeros16 = jnp.zeros((16,), jnp.float32)
    for r in range(16):
        for cc in range(H // 16):
            zero_v[r, pl.ds(cc * 16, 16)] = zeros16

    def zacc(i, carry):
        pltpu.sync_copy(zero_v, acc_sh.at[pl.ds((i * NS + s) * 16, 16)])
        return carry

    lax.fori_loop(0, RPT // 16, zacc, 0)
    plsc.subcore_barrier()

    pltpu.sync_copy(ei_hbm.at[0, wid, pl.ds(0, NBUF)], src_v.at[0])
    pltpu.sync_copy(ei_hbm.at[1, wid, pl.ds(0, NBUF)], dst_v.at[0])
    for b in range(NBUF):
        pltpu.async_copy(g_hbm.at[src_v.at[0, b]], rows_v.at[b], gsem)

    def group(gi, carry):
        p = lax.rem(gi, 2)
        pn = lax.rem(gi + 1, 2)
        notlast = gi + 1 < NGROUP
        jn = (gi + 1) * NBUF

        @pl.when(notlast)
        def _():
            pltpu.async_copy(ei_hbm.at[0, wid, pl.ds(jn, NBUF)], src_v.at[pn], isem)
            pltpu.async_copy(ei_hbm.at[1, wid, pl.ds(jn, NBUF)], dst_v.at[pn], isem)

        for b in range(NBUF):
            pltpu.make_async_copy(
                g_hbm.at[src_v.at[p, b]], rows_v.at[b], gsem
            ).wait()
            pltpu.async_copy(
                rows_v.at[b], acc_sh.at[dst_v.at[p, b]], ssem, add=True
            )

        @pl.when(notlast)
        def _():
            pltpu.make_async_copy(
                ei_hbm.at[0, wid, pl.ds(jn, NBUF)], src_v.at[pn], isem
            ).wait()
            pltpu.make_async_copy(
                ei_hbm.at[1, wid, pl.ds(jn, NBUF)], dst_v.at[pn], isem
            ).wait()

        for b in range(NBUF):
            pltpu.make_async_copy(
                rows_v.at[b], acc_sh.at[dst_v.at[p, b]], ssem
            ).wait()

            @pl.when(notlast)
            def _():
                pltpu.async_copy(g_hbm.at[src_v.at[pn, b]], rows_v.at[b], gsem)

        return carry

    lax.fori_loop(0, NGROUP, group, 0)

    for j in range(NGROUP * NBUF, NCHUNKS):
        pltpu.sync_copy(ei_hbm.at[0, wid, pl.ds(j, 1)], src_v.at[0, pl.ds(0, 1)])
        pltpu.sync_copy(ei_hbm.at[1, wid, pl.ds(j, 1)], dst_v.at[0, pl.ds(0, 1)])
        pltpu.async_copy(g_hbm.at[src_v.at[0, 0]], rows_v.at[0], gsem).wait()
        pltpu.async_copy(
            rows_v.at[0], acc_sh.at[dst_v.at[0, 0]], ssem, add=True
        ).wait()
    plsc.subcore_barrier()

    row0 = s * RPT
    pltpu.sync_copy(acc_sh.at[pl.ds(row0, RPT)], out_hbm.at[c, pl.ds(row0, RPT)])


@functools.cache
def _conv_kernel():
    return pl.kernel(
        _conv_body,
        out_type=jax.ShapeDtypeStruct((NC, NP, H), jnp.float32),
        mesh=_mesh(),
        scratch_types=[
            pltpu.VMEM_SHARED((NP, H), jnp.float32),
            pltpu.VMEM((2, NBUF, CHUNK), jnp.int32),
            pltpu.VMEM((2, NBUF, CHUNK), jnp.int32),
            pltpu.VMEM((NBUF, CHUNK, H), jnp.float32),
            pltpu.VMEM((16, H), jnp.float32),
            pltpu.SemaphoreType.DMA,
            pltpu.SemaphoreType.DMA,
            pltpu.SemaphoreType.DMA,
        ],
        compiler_params=pltpu.CompilerParams(
            use_tc_tiling_on_sc=False, skip_device_barrier=True
        ),
    )


def _dinv_body(degp_ref, dinv_ref):
    deg = jnp.sum(degp_ref[...], axis=0) + 1.0
    dinv_ref[...] = lax.rsqrt(deg)[:, None]


_dinv_kernel = pl.pallas_call(
    _dinv_body,
    grid=(NP // 1024,),
    in_specs=[pl.BlockSpec((NW, 1024), lambda i: (0, i))],
    out_specs=[pl.BlockSpec((1024, 1), lambda i: (i, 0))],
    out_shape=[jax.ShapeDtypeStruct((NP, 1), jnp.float32)],
)


def _layer_norm_tc(t, g, b):
    mu = jnp.mean(t, axis=-1, keepdims=True)
    v = jnp.mean((t - mu) ** 2, axis=-1, keepdims=True)
    return (t - mu) / jnp.sqrt(v + 1e-5) * g + b


def _stage_a_body(dinv_ref, x_ref, wlin_ref, blin_ref, wc1_ref, g1_ref):
    dinv = dinv_ref[...]
    h0 = jnp.maximum(x_ref[...] @ wlin_ref[...] + blin_ref[...], 0.0)
    g1_ref[...] = (h0 @ wc1_ref[...]) * dinv


_stage_a = pl.pallas_call(
    _stage_a_body,
    grid=(GRID,),
    in_specs=[
        pl.BlockSpec((BLK, 1), lambda i: (i, 0)),
        pl.BlockSpec((BLK, F), lambda i: (i, 0)),
        pl.BlockSpec((F, H), lambda i: (0, 0)),
        pl.BlockSpec((1, H), lambda i: (0, 0)),
        pl.BlockSpec((H, H), lambda i: (0, 0)),
    ],
    out_specs=[
        pl.BlockSpec((BLK, H), lambda i: (i, 0)),
    ],
    out_shape=[
        jax.ShapeDtypeStruct((N, H), jnp.float32),
    ],
)


def _stage_b_body(dinv_ref, acc_ref, g1_ref, bc1_ref, wc2_ref, g2_ref):
    dinv = dinv_ref[...]
    a = acc_ref[...]
    conv = dinv * (a[0] + a[1] + g1_ref[...]) + bc1_ref[...]
    h1 = jnp.maximum(conv, 0.0)
    g2_ref[...] = (h1 @ wc2_ref[...]) * dinv


_stage_b = pl.pallas_call(
    _stage_b_body,
    grid=(GRID,),
    in_specs=[
        pl.BlockSpec((BLK, 1), lambda i: (i, 0)),
        pl.BlockSpec((NC, BLK, H), lambda i: (0, i, 0)),
        pl.BlockSpec((BLK, H), lambda i: (i, 0)),
        pl.BlockSpec((1, H), lambda i: (0, 0)),
        pl.BlockSpec((H, H), lambda i: (0, 0)),
    ],
    out_specs=[
        pl.BlockSpec((BLK, H), lambda i: (i, 0)),
    ],
    out_shape=[
        jax.ShapeDtypeStruct((N, H), jnp.float32),
    ],
)


def _stage_c_body(dinv_ref, acc_ref, g2m_ref, bc2_ref, wih_ref, bih_ref, bhh_ref,
                  wp1_ref, bp1_ref, g1_ref, bn1_ref,
                  wp2_ref, bp2_ref, g2_ref, bn2_ref,
                  wp3_ref, bp3_ref, y_ref):
    dinv = dinv_ref[...]
    a = acc_ref[...]
    conv = dinv * (a[0] + a[1] + g2m_ref[...]) + bc2_ref[...]
    h2 = jnp.maximum(conv, 0.0)
    gi = h2 @ wih_ref[...] + bih_ref[...]
    bhh = bhh_ref[...]
    r = jax.nn.sigmoid(gi[:, :H] + bhh[:, :H])
    z = jax.nn.sigmoid(gi[:, H:2 * H] + bhh[:, H:2 * H])
    n = jnp.tanh(gi[:, 2 * H:] + r * bhh[:, 2 * H:])
    hcur = (1.0 - z) * n
    t1 = jnp.maximum(
        _layer_norm_tc(hcur @ wp1_ref[...] + bp1_ref[...], g1_ref[...], bn1_ref[...]), 0.0)
    t2 = jnp.maximum(
        _layer_norm_tc(t1 @ wp2_ref[...] + bp2_ref[...], g2_ref[...], bn2_ref[...]), 0.0)
    y_ref[...] = t2 @ wp3_ref[...] + bp3_ref[...]


_stage_c = pl.pallas_call(
    _stage_c_body,
    grid=(GRID,),
    in_specs=[
        pl.BlockSpec((BLK, 1), lambda i: (i, 0)),
        pl.BlockSpec((NC, BLK, H), lambda i: (0, i, 0)),
        pl.BlockSpec((BLK, H), lambda i: (i, 0)),
        pl.BlockSpec((1, H), lambda i: (0, 0)),
        pl.BlockSpec((H, 3 * H), lambda i: (0, 0)),
        pl.BlockSpec((1, 3 * H), lambda i: (0, 0)),
        pl.BlockSpec((1, 3 * H), lambda i: (0, 0)),
        pl.BlockSpec((H, H), lambda i: (0, 0)),
        pl.BlockSpec((1, H), lambda i: (0, 0)),
        pl.BlockSpec((1, H), lambda i: (0, 0)),
        pl.BlockSpec((1, H), lambda i: (0, 0)),
        pl.BlockSpec((H, H), lambda i: (0, 0)),
        pl.BlockSpec((1, H), lambda i: (0, 0)),
        pl.BlockSpec((1, H), lambda i: (0, 0)),
        pl.BlockSpec((1, H), lambda i: (0, 0)),
        pl.BlockSpec((H, 1), lambda i: (0, 0)),
        pl.BlockSpec((1, 1), lambda i: (0, 0)),
    ],
    out_specs=[pl.BlockSpec((BLK, 1), lambda i: (i, 0))],
    out_shape=[jax.ShapeDtypeStruct((N, 1), jnp.float32)],
)


def kernel(x, edge_index, W_lin, b_lin, Wc1, bc1, Wc2, bc2, W_ih, W_hh, b_ih, b_hh,
           Wp1, bp1, g1, bn1, Wp2, bp2, g2, bn2, Wp3, bp3):
    ei = edge_index.reshape(2, NW, NCHUNKS, CHUNK)

    degp = _deg_kernel()(ei)
    (dinv,) = _dinv_kernel(degp)
    (g1m,) = _stage_a(dinv, x, W_lin, b_lin.reshape(1, H), Wc1)
    acc1 = _conv_kernel()(g1m, ei)
    (g2m,) = _stage_b(dinv, acc1, g1m, bc1.reshape(1, H), Wc2)
    acc2 = _conv_kernel()(g2m, ei)
    (y,) = _stage_c(
        dinv, acc2, g2m, bc2.reshape(1, H), W_ih, b_ih.reshape(1, 3 * H),
        b_hh.reshape(1, 3 * H),
        Wp1, bp1.reshape(1, H), g1.reshape(1, H), bn1.reshape(1, H),
        Wp2, bp2.reshape(1, H), g2.reshape(1, H), bn2.reshape(1, H),
        Wp3, bp3.reshape(1, 1),
    )
    return y


# speedup vs baseline: 32.7922x; 1.0106x over previous
import functools

import jax
import jax.numpy as jnp
from jax import lax
from jax.experimental import pallas as pl
from jax.experimental.pallas import tpu as pltpu
from jax.experimental.pallas import tpu_sc as plsc

N = 10000
E = 320000
F = 128
H = 128

NP = 10240
NC = 2
NS = 16
NW = NC * NS
EPW = E // NW
CHUNK = 80
NCHUNKS = EPW // CHUNK
NBUF = 4
NGROUP = NCHUNKS // NBUF
RPT = NP // NS
BLK = 1000
GRID = N // BLK

@functools.cache
def _mesh():
    return plsc.VectorSubcoreMesh(
        core_axis_name="c", subcore_axis_name="s", num_cores=NC, num_subcores=NS
    )


def _deg_body(ei_hbm, out_hbm, deg_v, idx_v):
    c = lax.axis_index("c")
    s = lax.axis_index("s")
    wid = s * NC + c

    zeros16 = jnp.zeros((16,), jnp.float32)

    def zloop(i, carry):
        deg_v[pl.ds(i * 16, 16)] = zeros16
        return carry

    lax.fori_loop(0, NP // 16, zloop, 0)

    pltpu.sync_copy(ei_hbm.at[1, wid], idx_v)
    ones16 = jnp.ones((16,), jnp.float32)

    def body(i, carry):
        jc = i // (CHUNK // 16)
        k = lax.rem(i, CHUNK // 16)
        idx = idx_v[jc, pl.ds(k * 16, 16)]
        plsc.addupdate_scatter(deg_v, [idx], ones16)
        return carry

    lax.fori_loop(0, EPW // 16, body, 0)
    pltpu.sync_copy(deg_v, out_hbm.at[wid])


@functools.cache
def _deg_kernel():
    return pl.kernel(
        _deg_body,
        out_type=jax.ShapeDtypeStruct((NW, NP), jnp.float32),
        mesh=_mesh(),
        scratch_types=[
            pltpu.VMEM((NP,), jnp.float32),
            pltpu.VMEM((NCHUNKS, CHUNK), jnp.int32),
        ],
        compiler_params=pltpu.CompilerParams(
            needs_layout_passes=False, use_tc_tiling_on_sc=False
        ),
    )


def _conv_body(g_hbm, ei_hbm, out_hbm,
               acc_sh, src_v, dst_v, rows_v, zero_v, gsem, ssem, isem):
    c = lax.axis_index("c")
    s = lax.axis_index("s")
    wid = s * NC + c

    pltpu.sync_copy(ei_hbm.at[0, wid, pl.ds(0, NBUF)], src_v.at[0])
    pltpu.sync_copy(ei_hbm.at[1, wid, pl.ds(0, NBUF)], dst_v.at[0])
    for b in range(NBUF):
        pltpu.async_copy(g_hbm.at[src_v.at[0, b]], rows_v.at[b], gsem)

    zeros16 = jnp.zeros((16,), jnp.float32)
    for r in range(16):
        for cc in range(H // 16):
            zero_v[r, pl.ds(cc * 16, 16)] = zeros16

    def zacc(i, carry):
        pltpu.sync_copy(zero_v, acc_sh.at[pl.ds((i * NS + s) * 16, 16)])
        return carry

    lax.fori_loop(0, RPT // 16, zacc, 0)
    plsc.subcore_barrier()

    def group(gi, carry):
        p = lax.rem(gi, 2)
        pn = lax.rem(gi + 1, 2)
        notlast = gi + 1 < NGROUP
        jn = (gi + 1) * NBUF

        @pl.when(notlast)
        def _():
            pltpu.async_copy(ei_hbm.at[0, wid, pl.ds(jn, NBUF)], src_v.at[pn], isem)
            pltpu.async_copy(ei_hbm.at[1, wid, pl.ds(jn, NBUF)], dst_v.at[pn], isem)

        for b in range(NBUF):
            pltpu.make_async_copy(
                g_hbm.at[src_v.at[p, b]], rows_v.at[b], gsem
            ).wait()
            pltpu.async_copy(
                rows_v.at[b], acc_sh.at[dst_v.at[p, b]], ssem, add=True
            )

        @pl.when(notlast)
        def _():
            pltpu.make_async_copy(
                ei_hbm.at[0, wid, pl.ds(jn, NBUF)], src_v.at[pn], isem
            ).wait()
            pltpu.make_async_copy(
                ei_hbm.at[1, wid, pl.ds(jn, NBUF)], dst_v.at[pn], isem
            ).wait()

        for b in range(NBUF):
            pltpu.make_async_copy(
                rows_v.at[b], acc_sh.at[dst_v.at[p, b]], ssem
            ).wait()

            @pl.when(notlast)
            def _():
                pltpu.async_copy(g_hbm.at[src_v.at[pn, b]], rows_v.at[b], gsem)

        return carry

    lax.fori_loop(0, NGROUP, group, 0)

    for j in range(NGROUP * NBUF, NCHUNKS):
        pltpu.sync_copy(ei_hbm.at[0, wid, pl.ds(j, 1)], src_v.at[0, pl.ds(0, 1)])
        pltpu.sync_copy(ei_hbm.at[1, wid, pl.ds(j, 1)], dst_v.at[0, pl.ds(0, 1)])
        pltpu.async_copy(g_hbm.at[src_v.at[0, 0]], rows_v.at[0], gsem).wait()
        pltpu.async_copy(
            rows_v.at[0], acc_sh.at[dst_v.at[0, 0]], ssem, add=True
        ).wait()
    plsc.subcore_barrier()

    row0 = s * RPT
    pltpu.sync_copy(acc_sh.at[pl.ds(row0, RPT)], out_hbm.at[c, pl.ds(row0, RPT)])


@functools.cache
def _conv_kernel():
    return pl.kernel(
        _conv_body,
        out_type=jax.ShapeDtypeStruct((NC, NP, H), jnp.float32),
        mesh=_mesh(),
        scratch_types=[
            pltpu.VMEM_SHARED((NP, H), jnp.float32),
            pltpu.VMEM((2, NBUF, CHUNK), jnp.int32),
            pltpu.VMEM((2, NBUF, CHUNK), jnp.int32),
            pltpu.VMEM((NBUF, CHUNK, H), jnp.float32),
            pltpu.VMEM((16, H), jnp.float32),
            pltpu.SemaphoreType.DMA,
            pltpu.SemaphoreType.DMA,
            pltpu.SemaphoreType.DMA,
        ],
        compiler_params=pltpu.CompilerParams(use_tc_tiling_on_sc=False),
    )


def _dinv_body(degp_ref, dinv_ref):
    deg = jnp.sum(degp_ref[...], axis=0) + 1.0
    dinv_ref[...] = lax.rsqrt(deg)[:, None]


_dinv_kernel = pl.pallas_call(
    _dinv_body,
    grid=(NP // 1024,),
    in_specs=[pl.BlockSpec((NW, 1024), lambda i: (0, i))],
    out_specs=[pl.BlockSpec((1024, 1), lambda i: (i, 0))],
    out_shape=[jax.ShapeDtypeStruct((NP, 1), jnp.float32)],
)


def _layer_norm_tc(t, g, b):
    mu = jnp.mean(t, axis=-1, keepdims=True)
    v = jnp.mean((t - mu) ** 2, axis=-1, keepdims=True)
    return (t - mu) / jnp.sqrt(v + 1e-5) * g + b


def _stage_a_body(dinv_ref, x_ref, wlin_ref, blin_ref, wc1_ref, g1_ref):
    dinv = dinv_ref[...]
    h0 = jnp.maximum(x_ref[...] @ wlin_ref[...] + blin_ref[...], 0.0)
    g1_ref[...] = (h0 @ wc1_ref[...]) * dinv


_stage_a = pl.pallas_call(
    _stage_a_body,
    grid=(GRID,),
    in_specs=[
        pl.BlockSpec((BLK, 1), lambda i: (i, 0)),
        pl.BlockSpec((BLK, F), lambda i: (i, 0)),
        pl.BlockSpec((F, H), lambda i: (0, 0)),
        pl.BlockSpec((1, H), lambda i: (0, 0)),
        pl.BlockSpec((H, H), lambda i: (0, 0)),
    ],
    out_specs=[
        pl.BlockSpec((BLK, H), lambda i: (i, 0)),
    ],
    out_shape=[
        jax.ShapeDtypeStruct((N, H), jnp.float32),
    ],
)


def _stage_b_body(dinv_ref, acc_ref, g1_ref, bc1_ref, wc2_ref, g2_ref):
    dinv = dinv_ref[...]
    a = acc_ref[...]
    conv = dinv * (a[0] + a[1] + g1_ref[...]) + bc1_ref[...]
    h1 = jnp.maximum(conv, 0.0)
    g2_ref[...] = (h1 @ wc2_ref[...]) * dinv


_stage_b = pl.pallas_call(
    _stage_b_body,
    grid=(GRID,),
    in_specs=[
        pl.BlockSpec((BLK, 1), lambda i: (i, 0)),
        pl.BlockSpec((NC, BLK, H), lambda i: (0, i, 0)),
        pl.BlockSpec((BLK, H), lambda i: (i, 0)),
        pl.BlockSpec((1, H), lambda i: (0, 0)),
        pl.BlockSpec((H, H), lambda i: (0, 0)),
    ],
    out_specs=[
        pl.BlockSpec((BLK, H), lambda i: (i, 0)),
    ],
    out_shape=[
        jax.ShapeDtypeStruct((N, H), jnp.float32),
    ],
)


def _stage_c_body(dinv_ref, acc_ref, g2m_ref, bc2_ref, wih_ref, bih_ref, bhh_ref,
                  wp1_ref, bp1_ref, g1_ref, bn1_ref,
                  wp2_ref, bp2_ref, g2_ref, bn2_ref,
                  wp3_ref, bp3_ref, y_ref):
    dinv = dinv_ref[...]
    a = acc_ref[...]
    conv = dinv * (a[0] + a[1] + g2m_ref[...]) + bc2_ref[...]
    h2 = jnp.maximum(conv, 0.0)
    gi = h2 @ wih_ref[...] + bih_ref[...]
    bhh = bhh_ref[...]
    r = jax.nn.sigmoid(gi[:, :H] + bhh[:, :H])
    z = jax.nn.sigmoid(gi[:, H:2 * H] + bhh[:, H:2 * H])
    n = jnp.tanh(gi[:, 2 * H:] + r * bhh[:, 2 * H:])
    hcur = (1.0 - z) * n
    t1 = jnp.maximum(
        _layer_norm_tc(hcur @ wp1_ref[...] + bp1_ref[...], g1_ref[...], bn1_ref[...]), 0.0)
    t2 = jnp.maximum(
        _layer_norm_tc(t1 @ wp2_ref[...] + bp2_ref[...], g2_ref[...], bn2_ref[...]), 0.0)
    y_ref[...] = t2 @ wp3_ref[...] + bp3_ref[...]


_stage_c = pl.pallas_call(
    _stage_c_body,
    grid=(GRID,),
    in_specs=[
        pl.BlockSpec((BLK, 1), lambda i: (i, 0)),
        pl.BlockSpec((NC, BLK, H), lambda i: (0, i, 0)),
        pl.BlockSpec((BLK, H), lambda i: (i, 0)),
        pl.BlockSpec((1, H), lambda i: (0, 0)),
        pl.BlockSpec((H, 3 * H), lambda i: (0, 0)),
        pl.BlockSpec((1, 3 * H), lambda i: (0, 0)),
        pl.BlockSpec((1, 3 * H), lambda i: (0, 0)),
        pl.BlockSpec((H, H), lambda i: (0, 0)),
        pl.BlockSpec((1, H), lambda i: (0, 0)),
        pl.BlockSpec((1, H), lambda i: (0, 0)),
        pl.BlockSpec((1, H), lambda i: (0, 0)),
        pl.BlockSpec((H, H), lambda i: (0, 0)),
        pl.BlockSpec((1, H), lambda i: (0, 0)),
        pl.BlockSpec((1, H), lambda i: (0, 0)),
        pl.BlockSpec((1, H), lambda i: (0, 0)),
        pl.BlockSpec((H, 1), lambda i: (0, 0)),
        pl.BlockSpec((1, 1), lambda i: (0, 0)),
    ],
    out_specs=[pl.BlockSpec((BLK, 1), lambda i: (i, 0))],
    out_shape=[jax.ShapeDtypeStruct((N, 1), jnp.float32)],
)


def kernel(x, edge_index, W_lin, b_lin, Wc1, bc1, Wc2, bc2, W_ih, W_hh, b_ih, b_hh,
           Wp1, bp1, g1, bn1, Wp2, bp2, g2, bn2, Wp3, bp3):
    ei = edge_index.reshape(2, NW, NCHUNKS, CHUNK)

    degp = _deg_kernel()(ei)
    (dinv,) = _dinv_kernel(degp)
    (g1m,) = _stage_a(dinv, x, W_lin, b_lin.reshape(1, H), Wc1)
    acc1 = _conv_kernel()(g1m, ei)
    (g2m,) = _stage_b(dinv, acc1, g1m, bc1.reshape(1, H), Wc2)
    acc2 = _conv_kernel()(g2m, ei)
    (y,) = _stage_c(
        dinv, acc2, g2m, bc2.reshape(1, H), W_ih, b_ih.reshape(1, 3 * H),
        b_hh.reshape(1, 3 * H),
        Wp1, bp1.reshape(1, H), g1.reshape(1, H), bn1.reshape(1, H),
        Wp2, bp2.reshape(1, H), g2.reshape(1, H), bn2.reshape(1, H),
        Wp3, bp3.reshape(1, 1),
    )
    return y


# speedup vs baseline: 33.8871x; 1.0334x over previous
import functools

import jax
import jax.numpy as jnp
from jax import lax
from jax.experimental import pallas as pl
from jax.experimental.pallas import tpu as pltpu
from jax.experimental.pallas import tpu_sc as plsc

N = 10000
E = 320000
F = 128
H = 128

NP = 10240
NC = 2
NS = 16
NW = NC * NS
EPW = E // NW
CHUNK = 80
NCHUNKS = EPW // CHUNK
NBUF = 4
NGROUP = NCHUNKS // NBUF
RPT = NP // NS
BLK = 2000
GRID = N // BLK

@functools.cache
def _mesh():
    return plsc.VectorSubcoreMesh(
        core_axis_name="c", subcore_axis_name="s", num_cores=NC, num_subcores=NS
    )


def _deg_body(ei_hbm, out_hbm, deg_v, idx_v):
    c = lax.axis_index("c")
    s = lax.axis_index("s")
    wid = s * NC + c

    zeros16 = jnp.zeros((16,), jnp.float32)

    def zloop(i, carry):
        deg_v[pl.ds(i * 16, 16)] = zeros16
        return carry

    lax.fori_loop(0, NP // 16, zloop, 0)

    pltpu.sync_copy(ei_hbm.at[1, wid], idx_v)
    ones16 = jnp.ones((16,), jnp.float32)

    def body(i, carry):
        jc = i // (CHUNK // 16)
        k = lax.rem(i, CHUNK // 16)
        idx = idx_v[jc, pl.ds(k * 16, 16)]
        plsc.addupdate_scatter(deg_v, [idx], ones16)
        return carry

    lax.fori_loop(0, EPW // 16, body, 0)
    pltpu.sync_copy(deg_v, out_hbm.at[wid])


@functools.cache
def _deg_kernel():
    return pl.kernel(
        _deg_body,
        out_type=jax.ShapeDtypeStruct((NW, NP), jnp.float32),
        mesh=_mesh(),
        scratch_types=[
            pltpu.VMEM((NP,), jnp.float32),
            pltpu.VMEM((NCHUNKS, CHUNK), jnp.int32),
        ],
        compiler_params=pltpu.CompilerParams(
            needs_layout_passes=False, use_tc_tiling_on_sc=False
        ),
    )


def _conv_body(g_hbm, ei_hbm, out_hbm,
               acc_sh, src_v, dst_v, rows_v, zero_v, gsem, ssem, isem):
    c = lax.axis_index("c")
    s = lax.axis_index("s")
    wid = s * NC + c

    pltpu.sync_copy(ei_hbm.at[0, wid, pl.ds(0, NBUF)], src_v.at[0])
    pltpu.sync_copy(ei_hbm.at[1, wid, pl.ds(0, NBUF)], dst_v.at[0])
    for b in range(NBUF):
        pltpu.async_copy(g_hbm.at[src_v.at[0, b]], rows_v.at[b], gsem)

    zeros16 = jnp.zeros((16,), jnp.float32)
    for r in range(16):
        for cc in range(H // 16):
            zero_v[r, pl.ds(cc * 16, 16)] = zeros16

    def zacc(i, carry):
        pltpu.sync_copy(zero_v, acc_sh.at[pl.ds((i * NS + s) * 16, 16)])
        return carry

    lax.fori_loop(0, RPT // 16, zacc, 0)
    plsc.subcore_barrier()

    def group(gi, carry):
        p = lax.rem(gi, 2)
        pn = lax.rem(gi + 1, 2)
        notlast = gi + 1 < NGROUP
        jn = (gi + 1) * NBUF

        @pl.when(notlast)
        def _():
            pltpu.async_copy(ei_hbm.at[0, wid, pl.ds(jn, NBUF)], src_v.at[pn], isem)
            pltpu.async_copy(ei_hbm.at[1, wid, pl.ds(jn, NBUF)], dst_v.at[pn], isem)

        for b in range(NBUF):
            pltpu.make_async_copy(
                g_hbm.at[src_v.at[p, b]], rows_v.at[b], gsem
            ).wait()
            pltpu.async_copy(
                rows_v.at[b], acc_sh.at[dst_v.at[p, b]], ssem, add=True
            )

        @pl.when(notlast)
        def _():
            pltpu.make_async_copy(
                ei_hbm.at[0, wid, pl.ds(jn, NBUF)], src_v.at[pn], isem
            ).wait()
            pltpu.make_async_copy(
                ei_hbm.at[1, wid, pl.ds(jn, NBUF)], dst_v.at[pn], isem
            ).wait()

        for b in range(NBUF):
            pltpu.make_async_copy(
                rows_v.at[b], acc_sh.at[dst_v.at[p, b]], ssem
            ).wait()

            @pl.when(notlast)
            def _():
                pltpu.async_copy(g_hbm.at[src_v.at[pn, b]], rows_v.at[b], gsem)

        return carry

    lax.fori_loop(0, NGROUP, group, 0)

    for j in range(NGROUP * NBUF, NCHUNKS):
        pltpu.sync_copy(ei_hbm.at[0, wid, pl.ds(j, 1)], src_v.at[0, pl.ds(0, 1)])
        pltpu.sync_copy(ei_hbm.at[1, wid, pl.ds(j, 1)], dst_v.at[0, pl.ds(0, 1)])
        pltpu.async_copy(g_hbm.at[src_v.at[0, 0]], rows_v.at[0], gsem).wait()
        pltpu.async_copy(
            rows_v.at[0], acc_sh.at[dst_v.at[0, 0]], ssem, add=True
        ).wait()
    plsc.subcore_barrier()

    row0 = s * RPT
    pltpu.sync_copy(acc_sh.at[pl.ds(row0, RPT)], out_hbm.at[c, pl.ds(row0, RPT)])


@functools.cache
def _conv_kernel():
    return pl.kernel(
        _conv_body,
        out_type=jax.ShapeDtypeStruct((NC, NP, H), jnp.float32),
        mesh=_mesh(),
        scratch_types=[
            pltpu.VMEM_SHARED((NP, H), jnp.float32),
            pltpu.VMEM((2, NBUF, CHUNK), jnp.int32),
            pltpu.VMEM((2, NBUF, CHUNK), jnp.int32),
            pltpu.VMEM((NBUF, CHUNK, H), jnp.float32),
            pltpu.VMEM((16, H), jnp.float32),
            pltpu.SemaphoreType.DMA,
            pltpu.SemaphoreType.DMA,
            pltpu.SemaphoreType.DMA,
        ],
        compiler_params=pltpu.CompilerParams(use_tc_tiling_on_sc=False),
    )


def _dinv_body(degp_ref, dinv_ref):
    deg = jnp.sum(degp_ref[...], axis=0) + 1.0
    dinv_ref[...] = lax.rsqrt(deg)[:, None]


_dinv_kernel = pl.pallas_call(
    _dinv_body,
    grid=(NP // 1024,),
    in_specs=[pl.BlockSpec((NW, 1024), lambda i: (0, i))],
    out_specs=[pl.BlockSpec((1024, 1), lambda i: (i, 0))],
    out_shape=[jax.ShapeDtypeStruct((NP, 1), jnp.float32)],
)


def _layer_norm_tc(t, g, b):
    mu = jnp.mean(t, axis=-1, keepdims=True)
    v = jnp.mean((t - mu) ** 2, axis=-1, keepdims=True)
    return (t - mu) / jnp.sqrt(v + 1e-5) * g + b


def _stage_a_body(dinv_ref, x_ref, wlin_ref, blin_ref, wc1_ref, g1_ref):
    dinv = dinv_ref[...]
    h0 = jnp.maximum(x_ref[...] @ wlin_ref[...] + blin_ref[...], 0.0)
    g1_ref[...] = (h0 @ wc1_ref[...]) * dinv


_stage_a = pl.pallas_call(
    _stage_a_body,
    grid=(GRID,),
    in_specs=[
        pl.BlockSpec((BLK, 1), lambda i: (i, 0)),
        pl.BlockSpec((BLK, F), lambda i: (i, 0)),
        pl.BlockSpec((F, H), lambda i: (0, 0)),
        pl.BlockSpec((1, H), lambda i: (0, 0)),
        pl.BlockSpec((H, H), lambda i: (0, 0)),
    ],
    out_specs=[
        pl.BlockSpec((BLK, H), lambda i: (i, 0)),
    ],
    out_shape=[
        jax.ShapeDtypeStruct((N, H), jnp.float32),
    ],
)


def _stage_b_body(dinv_ref, acc_ref, g1_ref, bc1_ref, wc2_ref, g2_ref):
    dinv = dinv_ref[...]
    a = acc_ref[...]
    conv = dinv * (a[0] + a[1] + g1_ref[...]) + bc1_ref[...]
    h1 = jnp.maximum(conv, 0.0)
    g2_ref[...] = (h1 @ wc2_ref[...]) * dinv


_stage_b = pl.pallas_call(
    _stage_b_body,
    grid=(GRID,),
    in_specs=[
        pl.BlockSpec((BLK, 1), lambda i: (i, 0)),
        pl.BlockSpec((NC, BLK, H), lambda i: (0, i, 0)),
        pl.BlockSpec((BLK, H), lambda i: (i, 0)),
        pl.BlockSpec((1, H), lambda i: (0, 0)),
        pl.BlockSpec((H, H), lambda i: (0, 0)),
    ],
    out_specs=[
        pl.BlockSpec((BLK, H), lambda i: (i, 0)),
    ],
    out_shape=[
        jax.ShapeDtypeStruct((N, H), jnp.float32),
    ],
)


def _stage_c_body(dinv_ref, acc_ref, g2m_ref, bc2_ref, wih_ref, bih_ref, bhh_ref,
                  wp1_ref, bp1_ref, g1_ref, bn1_ref,
                  wp2_ref, bp2_ref, g2_ref, bn2_ref,
                  wp3_ref, bp3_ref, y_ref):
    dinv = dinv_ref[...]
    a = acc_ref[...]
    conv = dinv * (a[0] + a[1] + g2m_ref[...]) + bc2_ref[...]
    h2 = jnp.maximum(conv, 0.0)
    gi = h2 @ wih_ref[...] + bih_ref[...]
    bhh = bhh_ref[...]
    r = jax.nn.sigmoid(gi[:, :H] + bhh[:, :H])
    z = jax.nn.sigmoid(gi[:, H:2 * H] + bhh[:, H:2 * H])
    n = jnp.tanh(gi[:, 2 * H:] + r * bhh[:, 2 * H:])
    hcur = (1.0 - z) * n
    t1 = jnp.maximum(
        _layer_norm_tc(hcur @ wp1_ref[...] + bp1_ref[...], g1_ref[...], bn1_ref[...]), 0.0)
    t2 = jnp.maximum(
        _layer_norm_tc(t1 @ wp2_ref[...] + bp2_ref[...], g2_ref[...], bn2_ref[...]), 0.0)
    y_ref[...] = t2 @ wp3_ref[...] + bp3_ref[...]


_stage_c = pl.pallas_call(
    _stage_c_body,
    grid=(GRID,),
    in_specs=[
        pl.BlockSpec((BLK, 1), lambda i: (i, 0)),
        pl.BlockSpec((NC, BLK, H), lambda i: (0, i, 0)),
        pl.BlockSpec((BLK, H), lambda i: (i, 0)),
        pl.BlockSpec((1, H), lambda i: (0, 0)),
        pl.BlockSpec((H, 3 * H), lambda i: (0, 0)),
        pl.BlockSpec((1, 3 * H), lambda i: (0, 0)),
        pl.BlockSpec((1, 3 * H), lambda i: (0, 0)),
        pl.BlockSpec((H, H), lambda i: (0, 0)),
        pl.BlockSpec((1, H), lambda i: (0, 0)),
        pl.BlockSpec((1, H), lambda i: (0, 0)),
        pl.BlockSpec((1, H), lambda i: (0, 0)),
        pl.BlockSpec((H, H), lambda i: (0, 0)),
        pl.BlockSpec((1, H), lambda i: (0, 0)),
        pl.BlockSpec((1, H), lambda i: (0, 0)),
        pl.BlockSpec((1, H), lambda i: (0, 0)),
        pl.BlockSpec((H, 1), lambda i: (0, 0)),
        pl.BlockSpec((1, 1), lambda i: (0, 0)),
    ],
    out_specs=[pl.BlockSpec((BLK, 1), lambda i: (i, 0))],
    out_shape=[jax.ShapeDtypeStruct((N, 1), jnp.float32)],
)


def kernel(x, edge_index, W_lin, b_lin, Wc1, bc1, Wc2, bc2, W_ih, W_hh, b_ih, b_hh,
           Wp1, bp1, g1, bn1, Wp2, bp2, g2, bn2, Wp3, bp3):
    ei = edge_index.reshape(2, NW, NCHUNKS, CHUNK)

    degp = _deg_kernel()(ei)
    (dinv,) = _dinv_kernel(degp)
    (g1m,) = _stage_a(dinv, x, W_lin, b_lin.reshape(1, H), Wc1)
    acc1 = _conv_kernel()(g1m, ei)
    (g2m,) = _stage_b(dinv, acc1, g1m, bc1.reshape(1, H), Wc2)
    acc2 = _conv_kernel()(g2m, ei)
    (y,) = _stage_c(
        dinv, acc2, g2m, bc2.reshape(1, H), W_ih, b_ih.reshape(1, 3 * H),
        b_hh.reshape(1, 3 * H),
        Wp1, bp1.reshape(1, H), g1.reshape(1, H), bn1.reshape(1, H),
        Wp2, bp2.reshape(1, H), g2.reshape(1, H), bn2.reshape(1, H),
        Wp3, bp3.reshape(1, 1),
    )
    return y
